# baseline (speedup 1.0000x reference)
import functools

import jax
import jax.numpy as jnp
from jax import lax
from jax.experimental import pallas as pl
from jax.experimental.pallas import tpu as pltpu
from jax.experimental.pallas import tpu_sc as plsc

B = 256
NUM_WORKERS = 32


def _sc_gather_rows(table, idx, ch):
    m = idx.shape[0]
    _, d = table.shape
    per_w = m // NUM_WORKERS
    n_ch = per_w // ch
    assert per_w * NUM_WORKERS == m and n_ch * ch == per_w

    mesh = plsc.VectorSubcoreMesh(core_axis_name="c", subcore_axis_name="s")

    def body(table_hbm, idx_hbm, out_hbm, idx_v, buf, gsem):
        wid = lax.axis_index("s") * 2 + lax.axis_index("c")
        base = wid * per_w
        pltpu.sync_copy(idx_hbm.at[pl.ds(base, per_w)], idx_v)

        def chunk(i, carry):
            cp = pltpu.make_async_copy(
                table_hbm.at[idx_v.at[pl.ds(i * ch, ch)]], buf, gsem)
            cp.start()
            cp.wait()
            pltpu.sync_copy(buf, out_hbm.at[pl.ds(base + i * ch, ch)])
            return carry

        lax.fori_loop(0, n_ch, chunk, 0)

    f = pl.kernel(
        body,
        out_type=jax.ShapeDtypeStruct((m, d), jnp.float32),
        mesh=mesh,
        scratch_types=[
            pltpu.VMEM((per_w,), jnp.int32),
            pltpu.VMEM((ch, d), jnp.float32),
            pltpu.SemaphoreType.DMA,
        ],
    )
    return f(table, idx)


def _lstm_body(bo_ref, ms_ref, degc_ref, degr_ref, wih_ref, whh_ref,
               fcn_ref, bias_ref, xp_ref, y_ref, h_ref, c_ref, slab_ref, sem):
    b = pl.program_id(0)
    n_steps = ms_ref[b]
    h_ref[...] = jnp.zeros_like(h_ref)
    c_ref[...] = jnp.zeros_like(c_ref)
    degc = degc_ref[...]
    degr = degr_ref[0]
    bias = bias_ref[...]
    hdim = h_ref.shape[1]

    def step(t, s):
        cp = pltpu.make_async_copy(xp_ref.at[pl.ds(s, B)], slab_ref, sem)
        cp.start()
        cp.wait()
        xw = lax.dot_general(slab_ref[...], wih_ref[...],
                             (((1,), (1,)), ((), ())),
                             preferred_element_type=jnp.float32)
        hw = lax.dot_general(h_ref[...], whh_ref[...],
                             (((1,), (1,)), ((), ())),
                             preferred_element_type=jnp.float32)
        gates = xw + hw + bias
        gi = jax.nn.sigmoid(gates[:, 0:hdim])
        gf = jax.nn.sigmoid(gates[:, hdim:2 * hdim])
        gg = jnp.tanh(gates[:, 2 * hdim:3 * hdim])
        go = jax.nn.sigmoid(gates[:, 3 * hdim:4 * hdim])
        c_new = gf * c_ref[...] + gi * gg
        h_new = go * jnp.tanh(c_new)
        m = t < degc
        h_ref[...] = jnp.where(m, h_new, h_ref[...])
        c_ref[...] = jnp.where(m, c_new, c_ref[...])
        active = jnp.sum((t < degr).astype(jnp.int32))
        return s + active

    lax.fori_loop(0, n_steps, step, bo_ref[b])
    y_ref[...] = lax.dot_general(h_ref[...], fcn_ref[...],
                                 (((1,), (1,)), ((), ())),
                                 preferred_element_type=jnp.float32)


def _lstm_call(xp, bo, msteps, degc, degr, w_ih, w_hh, fcn, bias, npad):
    nb = npad // B
    h4 = w_ih.shape[0]
    d = w_ih.shape[1]
    hd = w_hh.shape[1]
    return pl.pallas_call(
        _lstm_body,
        grid=(nb,),
        in_specs=[
            pl.BlockSpec(memory_space=pltpu.SMEM),
            pl.BlockSpec(memory_space=pltpu.SMEM),
            pl.BlockSpec((B, 1), lambda b: (b, 0)),
            pl.BlockSpec((1, 1, B), lambda b: (b, 0, 0)),
            pl.BlockSpec((h4, d), lambda b: (0, 0)),
            pl.BlockSpec((h4, hd), lambda b: (0, 0)),
            pl.BlockSpec((hd, hd), lambda b: (0, 0)),
            pl.BlockSpec((1, h4), lambda b: (0, 0)),
            pl.BlockSpec(memory_space=pl.ANY),
        ],
        out_specs=pl.BlockSpec((B, hd), lambda b: (b, 0)),
        out_shape=jax.ShapeDtypeStruct((npad, hd), jnp.float32),
        scratch_shapes=[
            pltpu.VMEM((B, hd), jnp.float32),
            pltpu.VMEM((B, hd), jnp.float32),
            pltpu.VMEM((B, d), jnp.float32),
            pltpu.SemaphoreType.DMA,
        ],
    )(bo, msteps, degc, degr, w_ih, w_hh, fcn, bias, xp)


def _proj_body(x_ref, yun_ref, w_ref, b_ref, o_ref, *, act):
    o = lax.dot_general(x_ref[...], w_ref[...], (((1,), (1,)), ((), ())),
                        preferred_element_type=jnp.float32)
    o_ref[...] = act(o + b_ref[...] + yun_ref[...])


def _proj_call(xpad, yun, w, bias, act, npad):
    nb = npad // B
    d = w.shape[1]
    ho = w.shape[0]
    return pl.pallas_call(
        functools.partial(_proj_body, act=act),
        grid=(nb,),
        in_specs=[
            pl.BlockSpec((B, d), lambda b: (b, 0)),
            pl.BlockSpec((B, ho), lambda b: (b, 0)),
            pl.BlockSpec((ho, d), lambda b: (0, 0)),
            pl.BlockSpec((1, ho), lambda b: (0, 0)),
        ],
        out_specs=pl.BlockSpec((B, ho), lambda b: (b, 0)),
        out_shape=jax.ShapeDtypeStruct((npad, ho), jnp.float32),
    )(xpad, yun, w, bias)


def _prep(edge_index, n, npad, epad):
    src = edge_index[0]
    dst = edge_index[1]
    e = src.shape[0]
    order = jnp.argsort(dst)
    dst_s = dst[order]
    src_s = src[order]
    deg = jnp.bincount(dst, length=n).astype(jnp.int32)
    offsets = (jnp.cumsum(deg) - deg).astype(jnp.int32)
    t_e = jnp.arange(e, dtype=jnp.int32) - offsets[dst_s]

    perm = jnp.argsort(-deg)
    degp = jnp.zeros(npad, jnp.int32).at[:n].set(deg[perm])
    rank = jnp.zeros(n, jnp.int32).at[perm].set(jnp.arange(n, dtype=jnp.int32))
    cs = jnp.concatenate([jnp.zeros(1, jnp.int32),
                          jnp.cumsum(degp).astype(jnp.int32)])
    asc = degp[::-1]
    count_ge = (npad - jnp.searchsorted(asc, t_e, side="left")).astype(jnp.int32)
    r = rank[dst_s]
    blk = r // B
    c = jnp.clip(count_ge - blk * B, 0, B)
    p = (cs[blk * B] + t_e * c + (cs[(blk + 1) * B] - cs[blk * B + c])
         + (r - blk * B))
    src_packed = jnp.zeros(epad, jnp.int32).at[p].set(src_s)

    nb = npad // B
    bo = cs[jnp.arange(nb) * B]
    msteps = degp[jnp.arange(nb) * B]
    degc = degp.reshape(npad, 1)
    degr = degp.reshape(nb, 1, B)
    rankpad = jnp.zeros(npad, jnp.int32).at[:n].set(rank)
    return src_packed, bo, msteps, degc, degr, rankpad


def _layer(xpad, pre, w_ih, w_hh, b_ih, b_hh, w_self, b_self, w_neigh, act,
           npad):
    src_packed, bo, msteps, degc, degr, rankpad = pre
    hd = w_hh.shape[1]
    ho = w_self.shape[0]
    w_self_p = jnp.zeros((hd, w_self.shape[1]), jnp.float32).at[:ho].set(w_self)
    b_self_p = jnp.zeros((1, hd), jnp.float32).at[0, :ho].set(b_self)
    w_neigh_p = jnp.zeros((hd, w_neigh.shape[1]), jnp.float32).at[:ho].set(w_neigh)

    xp = _sc_gather_rows(xpad, src_packed, ch=128)
    bias = (b_ih + b_hh).reshape(1, -1)
    y_sorted = _lstm_call(xp, bo, msteps, degc, degr, w_ih, w_hh,
                          w_neigh_p, bias, npad)
    y_un = _sc_gather_rows(y_sorted, rankpad, ch=64)
    return _proj_call(xpad, y_un, w_self_p, b_self_p, act, npad)


def kernel(x, edge_index, W_ih1, W_hh1, b_ih1, b_hh1, fc_self_W1, fc_self_b1,
           fc_neigh_W1, W_ih2, W_hh2, b_ih2, b_hh2, fc_self_W2, fc_self_b2,
           fc_neigh_W2):
    n, d = x.shape
    e = edge_index.shape[1]
    npad = ((n + B - 1) // B) * B
    gran = NUM_WORKERS * 128
    epad = ((e + B + gran - 1) // gran) * gran

    pre = _prep(edge_index, n, npad, epad)
    xpad = jnp.zeros((npad, d), jnp.float32).at[:n].set(x)

    out1 = _layer(xpad, pre, W_ih1, W_hh1, b_ih1, b_hh1, fc_self_W1,
                  fc_self_b1, fc_neigh_W1, jax.nn.relu, npad)
    out2 = _layer(out1, pre, W_ih2, W_hh2, b_ih2, b_hh2, fc_self_W2,
                  fc_self_b2, fc_neigh_W2, jax.nn.sigmoid, npad)
    return out2[:n, :fc_self_W2.shape[0]]


# speedup vs baseline: 3.4430x; 3.4430x over previous
import functools

import jax
import jax.numpy as jnp
from jax import lax
from jax.experimental import pallas as pl
from jax.experimental.pallas import tpu as pltpu
from jax.experimental.pallas import tpu_sc as plsc

B = 256
NUM_WORKERS = 32


def _sc_gather_rows(table, idx, ch):
    m = idx.shape[0]
    _, d = table.shape
    per_w = m // NUM_WORKERS
    n_ch = per_w // ch
    assert per_w * NUM_WORKERS == m and n_ch * ch == per_w

    mesh = plsc.VectorSubcoreMesh(core_axis_name="c", subcore_axis_name="s")

    def body(table_hbm, idx_hbm, out_hbm, idx_v, buf, gsem):
        wid = lax.axis_index("s") * 2 + lax.axis_index("c")
        base = wid * per_w
        pltpu.sync_copy(idx_hbm.at[pl.ds(base, per_w)], idx_v)

        def chunk(i, carry):
            cp = pltpu.make_async_copy(
                table_hbm.at[idx_v.at[pl.ds(i * ch, ch)]], buf, gsem)
            cp.start()
            cp.wait()
            pltpu.sync_copy(buf, out_hbm.at[pl.ds(base + i * ch, ch)])
            return carry

        lax.fori_loop(0, n_ch, chunk, 0)

    f = pl.kernel(
        body,
        out_type=jax.ShapeDtypeStruct((m, d), jnp.float32),
        mesh=mesh,
        scratch_types=[
            pltpu.VMEM((per_w,), jnp.int32),
            pltpu.VMEM((ch, d), jnp.float32),
            pltpu.SemaphoreType.DMA,
        ],
    )
    return f(table, idx)


def _lstm_body(bo_ref, ms_ref, degc_ref, degr_ref, wih_ref, whh_ref,
               fcn_ref, bias_ref, xp_ref, y_ref, h_ref, c_ref, slab_ref, sem):
    b = pl.program_id(0)
    n_steps = ms_ref[b]
    h_ref[...] = jnp.zeros_like(h_ref)
    c_ref[...] = jnp.zeros_like(c_ref)
    degc = degc_ref[...]
    degr = degr_ref[0]
    bias = bias_ref[...]
    hdim = h_ref.shape[1]

    def step(t, s):
        cp = pltpu.make_async_copy(xp_ref.at[pl.ds(s, B)], slab_ref, sem)
        cp.start()
        cp.wait()
        xw = lax.dot_general(slab_ref[...], wih_ref[...],
                             (((1,), (1,)), ((), ())),
                             preferred_element_type=jnp.float32)
        hw = lax.dot_general(h_ref[...], whh_ref[...],
                             (((1,), (1,)), ((), ())),
                             preferred_element_type=jnp.float32)
        gates = xw + hw + bias
        gi = jax.nn.sigmoid(gates[:, 0:hdim])
        gf = jax.nn.sigmoid(gates[:, hdim:2 * hdim])
        gg = jnp.tanh(gates[:, 2 * hdim:3 * hdim])
        go = jax.nn.sigmoid(gates[:, 3 * hdim:4 * hdim])
        c_new = gf * c_ref[...] + gi * gg
        h_new = go * jnp.tanh(c_new)
        m = t < degc
        h_ref[...] = jnp.where(m, h_new, h_ref[...])
        c_ref[...] = jnp.where(m, c_new, c_ref[...])
        active = jnp.sum((t < degr).astype(jnp.int32))
        return s + active

    lax.fori_loop(0, n_steps, step, bo_ref[b])
    y_ref[...] = lax.dot_general(h_ref[...], fcn_ref[...],
                                 (((1,), (1,)), ((), ())),
                                 preferred_element_type=jnp.float32)


def _lstm_call(xp, bo, msteps, degc, degr, w_ih, w_hh, fcn, bias, npad):
    nb = npad // B
    h4 = w_ih.shape[0]
    d = w_ih.shape[1]
    hd = w_hh.shape[1]
    return pl.pallas_call(
        _lstm_body,
        grid=(nb,),
        in_specs=[
            pl.BlockSpec(memory_space=pltpu.SMEM),
            pl.BlockSpec(memory_space=pltpu.SMEM),
            pl.BlockSpec((B, 1), lambda b: (b, 0)),
            pl.BlockSpec((1, 1, B), lambda b: (b, 0, 0)),
            pl.BlockSpec((h4, d), lambda b: (0, 0)),
            pl.BlockSpec((h4, hd), lambda b: (0, 0)),
            pl.BlockSpec((hd, hd), lambda b: (0, 0)),
            pl.BlockSpec((1, h4), lambda b: (0, 0)),
            pl.BlockSpec(memory_space=pl.ANY),
        ],
        out_specs=pl.BlockSpec((B, hd), lambda b: (b, 0)),
        out_shape=jax.ShapeDtypeStruct((npad, hd), jnp.float32),
        scratch_shapes=[
            pltpu.VMEM((B, hd), jnp.float32),
            pltpu.VMEM((B, hd), jnp.float32),
            pltpu.VMEM((B, d), jnp.float32),
            pltpu.SemaphoreType.DMA,
        ],
    )(bo, msteps, degc, degr, w_ih, w_hh, fcn, bias, xp)


def _proj_body(x_ref, yun_ref, w_ref, b_ref, o_ref, *, act):
    o = lax.dot_general(x_ref[...], w_ref[...], (((1,), (1,)), ((), ())),
                        preferred_element_type=jnp.float32)
    o_ref[...] = act(o + b_ref[...] + yun_ref[...])


def _proj_call(xpad, yun, w, bias, act, npad):
    nb = npad // B
    d = w.shape[1]
    ho = w.shape[0]
    return pl.pallas_call(
        functools.partial(_proj_body, act=act),
        grid=(nb,),
        in_specs=[
            pl.BlockSpec((B, d), lambda b: (b, 0)),
            pl.BlockSpec((B, ho), lambda b: (b, 0)),
            pl.BlockSpec((ho, d), lambda b: (0, 0)),
            pl.BlockSpec((1, ho), lambda b: (0, 0)),
        ],
        out_specs=pl.BlockSpec((B, ho), lambda b: (b, 0)),
        out_shape=jax.ShapeDtypeStruct((npad, ho), jnp.float32),
    )(xpad, yun, w, bias)


def _prep(edge_index, n, npad, epad):
    src = edge_index[0]
    dst = edge_index[1]
    e = src.shape[0]
    deg = jnp.bincount(dst, length=n).astype(jnp.int32)
    offsets = (jnp.cumsum(deg) - deg).astype(jnp.int32)

    perm = jnp.argsort(-deg)
    degp = jnp.zeros(npad, jnp.int32).at[:n].set(deg[perm])
    rank = jnp.zeros(n, jnp.int32).at[perm].set(jnp.arange(n, dtype=jnp.int32))
    cs = jnp.concatenate([jnp.zeros(1, jnp.int32),
                          jnp.cumsum(degp).astype(jnp.int32)])
    hist = jnp.bincount(deg, length=e + 1).astype(jnp.int32)
    count_ge_tab = (n - jnp.cumsum(hist) + hist).astype(jnp.int32)

    blk_base = (rank // B) * B
    f2 = cs[blk_base] + (rank - blk_base) + cs[blk_base + B]
    node_tab = jnp.stack([offsets, rank, f2], axis=1)

    order = jnp.argsort(dst)
    es = jnp.concatenate([src[:, None], dst[:, None]], axis=1)[order]
    src_s = es[:, 0]
    nt = node_tab[es[:, 1]]
    t_e = jnp.arange(e, dtype=jnp.int32) - nt[:, 0]
    r = nt[:, 1]
    blkb = r - r % B
    c = jnp.clip(count_ge_tab[t_e] - blkb, 0, B)
    p = nt[:, 2] + t_e * c - cs[blkb + c]
    src_packed = jnp.zeros(epad, jnp.int32).at[p].set(src_s)

    nb = npad // B
    bo = cs[jnp.arange(nb) * B]
    msteps = degp[jnp.arange(nb) * B]
    degc = degp.reshape(npad, 1)
    degr = degp.reshape(nb, 1, B)
    rankpad = jnp.zeros(npad, jnp.int32).at[:n].set(rank)
    return src_packed, bo, msteps, degc, degr, rankpad


def _layer(xpad, pre, w_ih, w_hh, b_ih, b_hh, w_self, b_self, w_neigh, act,
           npad):
    src_packed, bo, msteps, degc, degr, rankpad = pre
    hd = w_hh.shape[1]
    ho = w_self.shape[0]
    w_self_p = jnp.zeros((hd, w_self.shape[1]), jnp.float32).at[:ho].set(w_self)
    b_self_p = jnp.zeros((1, hd), jnp.float32).at[0, :ho].set(b_self)
    w_neigh_p = jnp.zeros((hd, w_neigh.shape[1]), jnp.float32).at[:ho].set(w_neigh)

    xp = _sc_gather_rows(xpad, src_packed, ch=128)
    bias = (b_ih + b_hh).reshape(1, -1)
    y_sorted = _lstm_call(xp, bo, msteps, degc, degr, w_ih, w_hh,
                          w_neigh_p, bias, npad)
    y_un = _sc_gather_rows(y_sorted, rankpad, ch=64)
    return _proj_call(xpad, y_un, w_self_p, b_self_p, act, npad)


def kernel(x, edge_index, W_ih1, W_hh1, b_ih1, b_hh1, fc_self_W1, fc_self_b1,
           fc_neigh_W1, W_ih2, W_hh2, b_ih2, b_hh2, fc_self_W2, fc_self_b2,
           fc_neigh_W2):
    n, d = x.shape
    e = edge_index.shape[1]
    npad = ((n + B - 1) // B) * B
    gran = NUM_WORKERS * 128
    epad = ((e + B + gran - 1) // gran) * gran

    pre = _prep(edge_index, n, npad, epad)
    xpad = jnp.zeros((npad, d), jnp.float32).at[:n].set(x)

    out1 = _layer(xpad, pre, W_ih1, W_hh1, b_ih1, b_hh1, fc_self_W1,
                  fc_self_b1, fc_neigh_W1, jax.nn.relu, npad)
    out2 = _layer(out1, pre, W_ih2, W_hh2, b_ih2, b_hh2, fc_self_W2,
                  fc_self_b2, fc_neigh_W2, jax.nn.sigmoid, npad)
    return out2[:n, :fc_self_W2.shape[0]]


# speedup vs baseline: 6.5051x; 1.8894x over previous
import functools

import jax
import jax.numpy as jnp
from jax import lax
from jax.experimental import pallas as pl
from jax.experimental.pallas import tpu as pltpu
from jax.experimental.pallas import tpu_sc as plsc

B = 256
NUM_WORKERS = 32


def _sc_gather_rows(table, idx, ch):
    m = idx.shape[0]
    _, d = table.shape
    per_w = m // NUM_WORKERS
    n_ch = per_w // ch
    assert per_w * NUM_WORKERS == m and n_ch * ch == per_w

    mesh = plsc.VectorSubcoreMesh(core_axis_name="c", subcore_axis_name="s")

    def body(table_hbm, idx_hbm, out_hbm, idx_v, buf, gsem):
        wid = lax.axis_index("s") * 2 + lax.axis_index("c")
        base = wid * per_w
        pltpu.sync_copy(idx_hbm.at[pl.ds(base, per_w)], idx_v)

        def chunk(i, carry):
            cp = pltpu.make_async_copy(
                table_hbm.at[idx_v.at[pl.ds(i * ch, ch)]], buf, gsem)
            cp.start()
            cp.wait()
            pltpu.sync_copy(buf, out_hbm.at[pl.ds(base + i * ch, ch)])
            return carry

        lax.fori_loop(0, n_ch, chunk, 0)

    f = pl.kernel(
        body,
        out_type=jax.ShapeDtypeStruct((m, d), jnp.float32),
        mesh=mesh,
        scratch_types=[
            pltpu.VMEM((per_w,), jnp.int32),
            pltpu.VMEM((ch, d), jnp.float32),
            pltpu.SemaphoreType.DMA,
        ],
    )
    return f(table, idx)


def _sc_pack_rows(table, src_s, dst_s, off_n, rank_n, f2_n, cs, degp, e, epad):
    _, d = table.shape
    n = off_n.shape[0]
    npad1 = cs.shape[0]
    ch = 128
    per_w = epad // NUM_WORKERS
    n_ch = per_w // ch
    assert n_ch * ch * NUM_WORKERS == epad

    mesh = plsc.VectorSubcoreMesh(core_axis_name="c", subcore_axis_name="s")

    def body(table_hbm, src_hbm, dst_hbm, off_hbm, rank_hbm, f2_hbm, cs_hbm,
             degp_hbm, xp_hbm, src_v, dst_v, off_v, rank_v, f2_v, cs_v,
             degp_v, pbuf, rows, gsem, ssem):
        wid = lax.axis_index("s") * 2 + lax.axis_index("c")
        base = wid * per_w
        pltpu.sync_copy(src_hbm.at[pl.ds(base, per_w)], src_v)
        pltpu.sync_copy(dst_hbm.at[pl.ds(base, per_w)], dst_v)
        pltpu.sync_copy(off_hbm, off_v)
        pltpu.sync_copy(rank_hbm, rank_v)
        pltpu.sync_copy(f2_hbm, f2_v)
        pltpu.sync_copy(cs_hbm, cs_v)
        pltpu.sync_copy(degp_hbm, degp_v)

        lane16 = lax.iota(jnp.int32, 16)

        def chunk(i, carry):
            for v in range(ch // 16):
                o = i * ch + v * 16
                j = base + o + lane16
                dstv = dst_v[pl.ds(o, 16)]
                offv = plsc.load_gather(off_v, [dstv])
                rkv = plsc.load_gather(rank_v, [dstv])
                f2v = plsc.load_gather(f2_v, [dstv])
                t = j - offv
                blkb = rkv & ~(B - 1)
                lo = jnp.zeros((16,), jnp.int32)
                hi = jnp.full((16,), B, jnp.int32)
                for _ in range(8):
                    mid = (lo + hi) >> 1
                    dv = plsc.load_gather(degp_v, [blkb + mid])
                    ge = dv >= t
                    lo = jnp.where(ge, mid + 1, lo)
                    hi = jnp.where(ge, hi, mid)
                csv = plsc.load_gather(cs_v, [blkb + lo])
                p = f2v + t * lo - csv
                p = jnp.where(j < e, p, e + (j - e) % (epad - e))
                pbuf[pl.ds(v * 16, 16)] = p
            cp = pltpu.make_async_copy(
                table_hbm.at[src_v.at[pl.ds(i * ch, ch)]], rows, gsem)
            cp.start()
            cp.wait()
            sp = pltpu.make_async_copy(rows, xp_hbm.at[pbuf], ssem)
            sp.start()
            sp.wait()
            return carry

        lax.fori_loop(0, n_ch, chunk, 0)

    f = pl.kernel(
        body,
        out_type=jax.ShapeDtypeStruct((epad, d), jnp.float32),
        mesh=mesh,
        compiler_params=pltpu.CompilerParams(needs_layout_passes=False),
        scratch_types=[
            pltpu.VMEM((per_w,), jnp.int32),
            pltpu.VMEM((per_w,), jnp.int32),
            pltpu.VMEM((n,), jnp.int32),
            pltpu.VMEM((n,), jnp.int32),
            pltpu.VMEM((n,), jnp.int32),
            pltpu.VMEM((npad1,), jnp.int32),
            pltpu.VMEM((degp.shape[0],), jnp.int32),
            pltpu.VMEM((ch,), jnp.int32),
            pltpu.VMEM((ch, d), jnp.float32),
            pltpu.SemaphoreType.DMA,
            pltpu.SemaphoreType.DMA,
        ],
    )
    return f(table, src_s, dst_s, off_n, rank_n, f2_n, cs, degp)


def _lstm_body(bo_ref, ms_ref, degc_ref, degr_ref, wih_ref, whh_ref,
               fcn_ref, bias_ref, xp_ref, y_ref, h_ref, c_ref, slab_ref, sem):
    b = pl.program_id(0)
    n_steps = ms_ref[b]
    h_ref[...] = jnp.zeros_like(h_ref)
    c_ref[...] = jnp.zeros_like(c_ref)
    degc = degc_ref[...]
    degr = degr_ref[0]
    bias = bias_ref[...]
    hdim = h_ref.shape[1]

    def step(t, s):
        cp = pltpu.make_async_copy(xp_ref.at[pl.ds(s, B)], slab_ref, sem)
        cp.start()
        cp.wait()
        xw = lax.dot_general(slab_ref[...], wih_ref[...],
                             (((1,), (1,)), ((), ())),
                             preferred_element_type=jnp.float32)
        hw = lax.dot_general(h_ref[...], whh_ref[...],
                             (((1,), (1,)), ((), ())),
                             preferred_element_type=jnp.float32)
        gates = xw + hw + bias
        gi = jax.nn.sigmoid(gates[:, 0:hdim])
        gf = jax.nn.sigmoid(gates[:, hdim:2 * hdim])
        gg = jnp.tanh(gates[:, 2 * hdim:3 * hdim])
        go = jax.nn.sigmoid(gates[:, 3 * hdim:4 * hdim])
        c_new = gf * c_ref[...] + gi * gg
        h_new = go * jnp.tanh(c_new)
        m = t < degc
        h_ref[...] = jnp.where(m, h_new, h_ref[...])
        c_ref[...] = jnp.where(m, c_new, c_ref[...])
        active = jnp.sum((t < degr).astype(jnp.int32))
        return s + active

    lax.fori_loop(0, n_steps, step, bo_ref[b])
    y_ref[...] = lax.dot_general(h_ref[...], fcn_ref[...],
                                 (((1,), (1,)), ((), ())),
                                 preferred_element_type=jnp.float32)


def _lstm_call(xp, bo, msteps, degc, degr, w_ih, w_hh, fcn, bias, npad):
    nb = npad // B
    h4 = w_ih.shape[0]
    d = w_ih.shape[1]
    hd = w_hh.shape[1]
    return pl.pallas_call(
        _lstm_body,
        grid=(nb,),
        in_specs=[
            pl.BlockSpec(memory_space=pltpu.SMEM),
            pl.BlockSpec(memory_space=pltpu.SMEM),
            pl.BlockSpec((B, 1), lambda b: (b, 0)),
            pl.BlockSpec((1, 1, B), lambda b: (b, 0, 0)),
            pl.BlockSpec((h4, d), lambda b: (0, 0)),
            pl.BlockSpec((h4, hd), lambda b: (0, 0)),
            pl.BlockSpec((hd, hd), lambda b: (0, 0)),
            pl.BlockSpec((1, h4), lambda b: (0, 0)),
            pl.BlockSpec(memory_space=pl.ANY),
        ],
        out_specs=pl.BlockSpec((B, hd), lambda b: (b, 0)),
        out_shape=jax.ShapeDtypeStruct((npad, hd), jnp.float32),
        scratch_shapes=[
            pltpu.VMEM((B, hd), jnp.float32),
            pltpu.VMEM((B, hd), jnp.float32),
            pltpu.VMEM((B, d), jnp.float32),
            pltpu.SemaphoreType.DMA,
        ],
    )(bo, msteps, degc, degr, w_ih, w_hh, fcn, bias, xp)


def _proj_body(x_ref, yun_ref, w_ref, b_ref, o_ref, *, act):
    o = lax.dot_general(x_ref[...], w_ref[...], (((1,), (1,)), ((), ())),
                        preferred_element_type=jnp.float32)
    o_ref[...] = act(o + b_ref[...] + yun_ref[...])


def _proj_call(xpad, yun, w, bias, act, npad):
    nb = npad // B
    d = w.shape[1]
    ho = w.shape[0]
    return pl.pallas_call(
        functools.partial(_proj_body, act=act),
        grid=(nb,),
        in_specs=[
            pl.BlockSpec((B, d), lambda b: (b, 0)),
            pl.BlockSpec((B, ho), lambda b: (b, 0)),
            pl.BlockSpec((ho, d), lambda b: (0, 0)),
            pl.BlockSpec((1, ho), lambda b: (0, 0)),
        ],
        out_specs=pl.BlockSpec((B, ho), lambda b: (b, 0)),
        out_shape=jax.ShapeDtypeStruct((npad, ho), jnp.float32),
    )(xpad, yun, w, bias)


def _prep(edge_index, n, npad, epad):
    src = edge_index[0]
    dst = edge_index[1]
    e = src.shape[0]
    deg = jnp.bincount(dst, length=n).astype(jnp.int32)
    offsets = (jnp.cumsum(deg) - deg).astype(jnp.int32)

    perm = jnp.argsort(-deg)
    degp = jnp.zeros(npad, jnp.int32).at[:n].set(deg[perm])
    rank = jnp.zeros(n, jnp.int32).at[perm].set(jnp.arange(n, dtype=jnp.int32))
    cs = jnp.concatenate([jnp.zeros(1, jnp.int32),
                          jnp.cumsum(degp).astype(jnp.int32)])
    blk_base = (rank // B) * B
    f2 = (cs[blk_base] + (rank - blk_base) + cs[blk_base + B]).astype(jnp.int32)

    order = jnp.argsort(dst)
    es = jnp.concatenate([src[:, None], dst[:, None]], axis=1)[order]
    src_s = jnp.zeros(epad, jnp.int32).at[:e].set(es[:, 0])
    dst_s = jnp.zeros(epad, jnp.int32).at[:e].set(es[:, 1])

    nb = npad // B
    bo = cs[jnp.arange(nb) * B]
    msteps = degp[jnp.arange(nb) * B]
    degc = degp.reshape(npad, 1)
    degr = degp.reshape(nb, 1, B)
    rankpad = jnp.zeros(npad, jnp.int32).at[:n].set(rank)
    return (src_s, dst_s, offsets, rank, f2, cs, degp,
            bo, msteps, degc, degr, rankpad)


def _layer(xpad, pre, w_ih, w_hh, b_ih, b_hh, w_self, b_self, w_neigh, act,
           npad, e, epad):
    (src_s, dst_s, offsets, rank, f2, cs, degp,
     bo, msteps, degc, degr, rankpad) = pre
    hd = w_hh.shape[1]
    ho = w_self.shape[0]
    w_self_p = jnp.zeros((hd, w_self.shape[1]), jnp.float32).at[:ho].set(w_self)
    b_self_p = jnp.zeros((1, hd), jnp.float32).at[0, :ho].set(b_self)
    w_neigh_p = jnp.zeros((hd, w_neigh.shape[1]), jnp.float32).at[:ho].set(w_neigh)

    xp = _sc_pack_rows(xpad, src_s, dst_s, offsets, rank, f2, cs, degp,
                       e, epad)
    bias = (b_ih + b_hh).reshape(1, -1)
    y_sorted = _lstm_call(xp, bo, msteps, degc, degr, w_ih, w_hh,
                          w_neigh_p, bias, npad)
    y_un = _sc_gather_rows(y_sorted, rankpad, ch=64)
    return _proj_call(xpad, y_un, w_self_p, b_self_p, act, npad)


def kernel(x, edge_index, W_ih1, W_hh1, b_ih1, b_hh1, fc_self_W1, fc_self_b1,
           fc_neigh_W1, W_ih2, W_hh2, b_ih2, b_hh2, fc_self_W2, fc_self_b2,
           fc_neigh_W2):
    n, d = x.shape
    e = edge_index.shape[1]
    npad = ((n + B - 1) // B) * B
    gran = NUM_WORKERS * 128
    epad = ((e + B + gran - 1) // gran) * gran

    pre = _prep(edge_index, n, npad, epad)
    xpad = jnp.zeros((npad, d), jnp.float32).at[:n].set(x)

    out1 = _layer(xpad, pre, W_ih1, W_hh1, b_ih1, b_hh1, fc_self_W1,
                  fc_self_b1, fc_neigh_W1, jax.nn.relu, npad, e, epad)
    out2 = _layer(out1, pre, W_ih2, W_hh2, b_ih2, b_hh2, fc_self_W2,
                  fc_self_b2, fc_neigh_W2, jax.nn.sigmoid, npad, e, epad)
    return out2[:n, :fc_self_W2.shape[0]]


# speedup vs baseline: 8.5542x; 1.3150x over previous
import functools

import jax
import jax.numpy as jnp
from jax import lax
from jax.experimental import pallas as pl
from jax.experimental.pallas import tpu as pltpu
from jax.experimental.pallas import tpu_sc as plsc

B = 256
NUM_WORKERS = 32


def _sc_gather_rows(table, idx, ch):
    m = idx.shape[0]
    _, d = table.shape
    per_w = m // NUM_WORKERS
    n_ch = per_w // ch
    assert per_w * NUM_WORKERS == m and n_ch * ch == per_w

    mesh = plsc.VectorSubcoreMesh(core_axis_name="c", subcore_axis_name="s")

    def body(table_hbm, idx_hbm, out_hbm, idx_v, buf, gsem):
        wid = lax.axis_index("s") * 2 + lax.axis_index("c")
        base = wid * per_w
        pltpu.sync_copy(idx_hbm.at[pl.ds(base, per_w)], idx_v)

        def chunk(i, carry):
            cp = pltpu.make_async_copy(
                table_hbm.at[idx_v.at[pl.ds(i * ch, ch)]], buf, gsem)
            cp.start()
            cp.wait()
            pltpu.sync_copy(buf, out_hbm.at[pl.ds(base + i * ch, ch)])
            return carry

        lax.fori_loop(0, n_ch, chunk, 0)

    f = pl.kernel(
        body,
        out_type=jax.ShapeDtypeStruct((m, d), jnp.float32),
        mesh=mesh,
        scratch_types=[
            pltpu.VMEM((per_w,), jnp.int32),
            pltpu.VMEM((ch, d), jnp.float32),
            pltpu.SemaphoreType.DMA,
        ],
    )
    return f(table, idx)


def _sc_pack_rows(table, src_s, dst_s, off_n, rank_n, f2_n, cs, degp, e, epad):
    _, d = table.shape
    n = off_n.shape[0]
    npad1 = cs.shape[0]
    ch = 128
    per_w = epad // NUM_WORKERS
    n_ch = per_w // ch
    assert n_ch * ch * NUM_WORKERS == epad

    mesh = plsc.VectorSubcoreMesh(core_axis_name="c", subcore_axis_name="s")

    def body(table_hbm, src_hbm, dst_hbm, off_hbm, rank_hbm, f2_hbm, cs_hbm,
             degp_hbm, xp_hbm, src_v, dst_v, off_v, rank_v, f2_v, cs_v,
             degp_v, pbuf, rows, gsem, ssem):
        wid = lax.axis_index("s") * 2 + lax.axis_index("c")
        base = wid * per_w
        pltpu.sync_copy(src_hbm.at[pl.ds(base, per_w)], src_v)
        pltpu.sync_copy(dst_hbm.at[pl.ds(base, per_w)], dst_v)
        pltpu.sync_copy(off_hbm, off_v)
        pltpu.sync_copy(rank_hbm, rank_v)
        pltpu.sync_copy(f2_hbm, f2_v)
        pltpu.sync_copy(cs_hbm, cs_v)
        pltpu.sync_copy(degp_hbm, degp_v)

        lane16 = lax.iota(jnp.int32, 16)

        def chunk(i, carry):
            for v in range(ch // 16):
                o = i * ch + v * 16
                j = base + o + lane16
                dstv = dst_v[pl.ds(o, 16)]
                offv = plsc.load_gather(off_v, [dstv])
                rkv = plsc.load_gather(rank_v, [dstv])
                f2v = plsc.load_gather(f2_v, [dstv])
                t = j - offv
                blkb = rkv & ~(B - 1)
                lo = jnp.zeros((16,), jnp.int32)
                hi = jnp.full((16,), B, jnp.int32)
                for _ in range(8):
                    mid = (lo + hi) >> 1
                    dv = plsc.load_gather(degp_v, [blkb + mid])
                    ge = dv >= t
                    lo = jnp.where(ge, mid + 1, lo)
                    hi = jnp.where(ge, hi, mid)
                csv = plsc.load_gather(cs_v, [blkb + lo])
                p = f2v + t * lo - csv
                p = jnp.where(j < e, p, e + (j - e) % (epad - e))
                pbuf[pl.ds(v * 16, 16)] = p
            cp = pltpu.make_async_copy(
                table_hbm.at[src_v.at[pl.ds(i * ch, ch)]], rows, gsem)
            cp.start()
            cp.wait()
            sp = pltpu.make_async_copy(rows, xp_hbm.at[pbuf], ssem)
            sp.start()
            sp.wait()
            return carry

        lax.fori_loop(0, n_ch, chunk, 0)

    f = pl.kernel(
        body,
        out_type=jax.ShapeDtypeStruct((epad, d), jnp.float32),
        mesh=mesh,
        compiler_params=pltpu.CompilerParams(needs_layout_passes=False),
        scratch_types=[
            pltpu.VMEM((per_w,), jnp.int32),
            pltpu.VMEM((per_w,), jnp.int32),
            pltpu.VMEM((n,), jnp.int32),
            pltpu.VMEM((n,), jnp.int32),
            pltpu.VMEM((n,), jnp.int32),
            pltpu.VMEM((npad1,), jnp.int32),
            pltpu.VMEM((degp.shape[0],), jnp.int32),
            pltpu.VMEM((ch,), jnp.int32),
            pltpu.VMEM((ch, d), jnp.float32),
            pltpu.SemaphoreType.DMA,
            pltpu.SemaphoreType.DMA,
        ],
    )
    return f(table, src_s, dst_s, off_n, rank_n, f2_n, cs, degp)


def _lstm_body(bo_ref, ms_ref, degc_ref, degr_ref, wih_ref, whh_ref,
               fcn_ref, bias_ref, xp_ref, y_ref, h_ref, c_ref, slab_ref, sem):
    b = pl.program_id(0)
    n_steps = ms_ref[b]
    h_ref[...] = jnp.zeros_like(h_ref)
    c_ref[...] = jnp.zeros_like(c_ref)
    degc = degc_ref[...]
    degr = degr_ref[0]
    bias = bias_ref[...]
    hdim = h_ref.shape[1]
    s0 = bo_ref[b]

    @pl.when(n_steps > 0)
    def _():
        pltpu.make_async_copy(xp_ref.at[pl.ds(s0, B)], slab_ref.at[0],
                              sem.at[0]).start()

    def step(t, s):
        slot = lax.rem(t, 2)
        nslot = lax.rem(t + 1, 2)
        active = jnp.sum((t < degr).astype(jnp.int32))
        s_next = s + active

        @pl.when(t + 1 < n_steps)
        def _():
            pltpu.make_async_copy(xp_ref.at[pl.ds(s_next, B)],
                                  slab_ref.at[nslot], sem.at[nslot]).start()

        pltpu.make_async_copy(xp_ref.at[pl.ds(s, B)], slab_ref.at[slot],
                              sem.at[slot]).wait()
        xw = lax.dot_general(slab_ref[slot], wih_ref[...],
                             (((1,), (1,)), ((), ())),
                             preferred_element_type=jnp.float32)
        hw = lax.dot_general(h_ref[...], whh_ref[...],
                             (((1,), (1,)), ((), ())),
                             preferred_element_type=jnp.float32)
        gates = xw + hw + bias
        gi = jax.nn.sigmoid(gates[:, 0:hdim])
        gf = jax.nn.sigmoid(gates[:, hdim:2 * hdim])
        gg = jnp.tanh(gates[:, 2 * hdim:3 * hdim])
        go = jax.nn.sigmoid(gates[:, 3 * hdim:4 * hdim])
        c_new = gf * c_ref[...] + gi * gg
        h_new = go * jnp.tanh(c_new)
        m = t < degc
        h_ref[...] = jnp.where(m, h_new, h_ref[...])
        c_ref[...] = jnp.where(m, c_new, c_ref[...])
        return s_next

    lax.fori_loop(0, n_steps, step, s0)
    y_ref[...] = lax.dot_general(h_ref[...], fcn_ref[...],
                                 (((1,), (1,)), ((), ())),
                                 preferred_element_type=jnp.float32)


def _lstm_call(xp, bo, msteps, degc, degr, w_ih, w_hh, fcn, bias, npad):
    nb = npad // B
    h4 = w_ih.shape[0]
    d = w_ih.shape[1]
    hd = w_hh.shape[1]
    return pl.pallas_call(
        _lstm_body,
        grid=(nb,),
        in_specs=[
            pl.BlockSpec(memory_space=pltpu.SMEM),
            pl.BlockSpec(memory_space=pltpu.SMEM),
            pl.BlockSpec((B, 1), lambda b: (b, 0)),
            pl.BlockSpec((1, 1, B), lambda b: (b, 0, 0)),
            pl.BlockSpec((h4, d), lambda b: (0, 0)),
            pl.BlockSpec((h4, hd), lambda b: (0, 0)),
            pl.BlockSpec((hd, hd), lambda b: (0, 0)),
            pl.BlockSpec((1, h4), lambda b: (0, 0)),
            pl.BlockSpec(memory_space=pl.ANY),
        ],
        out_specs=pl.BlockSpec((B, hd), lambda b: (b, 0)),
        out_shape=jax.ShapeDtypeStruct((npad, hd), jnp.float32),
        scratch_shapes=[
            pltpu.VMEM((B, hd), jnp.float32),
            pltpu.VMEM((B, hd), jnp.float32),
            pltpu.VMEM((2, B, d), jnp.float32),
            pltpu.SemaphoreType.DMA((2,)),
        ],
    )(bo, msteps, degc, degr, w_ih, w_hh, fcn, bias, xp)


def _proj_body(x_ref, yun_ref, w_ref, b_ref, o_ref, *, act):
    o = lax.dot_general(x_ref[...], w_ref[...], (((1,), (1,)), ((), ())),
                        preferred_element_type=jnp.float32)
    o_ref[...] = act(o + b_ref[...] + yun_ref[...])


def _proj_call(xpad, yun, w, bias, act, npad):
    nb = npad // B
    d = w.shape[1]
    ho = w.shape[0]
    return pl.pallas_call(
        functools.partial(_proj_body, act=act),
        grid=(nb,),
        in_specs=[
            pl.BlockSpec((B, d), lambda b: (b, 0)),
            pl.BlockSpec((B, ho), lambda b: (b, 0)),
            pl.BlockSpec((ho, d), lambda b: (0, 0)),
            pl.BlockSpec((1, ho), lambda b: (0, 0)),
        ],
        out_specs=pl.BlockSpec((B, ho), lambda b: (b, 0)),
        out_shape=jax.ShapeDtypeStruct((npad, ho), jnp.float32),
    )(xpad, yun, w, bias)


def _prep(edge_index, n, npad, epad):
    src = edge_index[0]
    dst = edge_index[1]
    e = src.shape[0]
    deg = jnp.bincount(dst, length=n).astype(jnp.int32)
    offsets = (jnp.cumsum(deg) - deg).astype(jnp.int32)

    perm = jnp.argsort(-deg)
    degp = jnp.zeros(npad, jnp.int32).at[:n].set(deg[perm])
    rank = jnp.zeros(n, jnp.int32).at[perm].set(jnp.arange(n, dtype=jnp.int32))
    cs = jnp.concatenate([jnp.zeros(1, jnp.int32),
                          jnp.cumsum(degp).astype(jnp.int32)])
    blk_base = (rank // B) * B
    f2 = (cs[blk_base] + (rank - blk_base) + cs[blk_base + B]).astype(jnp.int32)

    order = jnp.argsort(dst)
    es = jnp.concatenate([src[:, None], dst[:, None]], axis=1)[order]
    src_s = jnp.zeros(epad, jnp.int32).at[:e].set(es[:, 0])
    dst_s = jnp.zeros(epad, jnp.int32).at[:e].set(es[:, 1])

    nb = npad // B
    bo = cs[jnp.arange(nb) * B]
    msteps = degp[jnp.arange(nb) * B]
    degc = degp.reshape(npad, 1)
    degr = degp.reshape(nb, 1, B)
    rankpad = jnp.zeros(npad, jnp.int32).at[:n].set(rank)
    return (src_s, dst_s, offsets, rank, f2, cs, degp,
            bo, msteps, degc, degr, rankpad)


def _layer(xpad, pre, w_ih, w_hh, b_ih, b_hh, w_self, b_self, w_neigh, act,
           npad, e, epad):
    (src_s, dst_s, offsets, rank, f2, cs, degp,
     bo, msteps, degc, degr, rankpad) = pre
    hd = w_hh.shape[1]
    ho = w_self.shape[0]
    w_self_p = jnp.zeros((hd, w_self.shape[1]), jnp.float32).at[:ho].set(w_self)
    b_self_p = jnp.zeros((1, hd), jnp.float32).at[0, :ho].set(b_self)
    w_neigh_p = jnp.zeros((hd, w_neigh.shape[1]), jnp.float32).at[:ho].set(w_neigh)

    xp = _sc_pack_rows(xpad, src_s, dst_s, offsets, rank, f2, cs, degp,
                       e, epad)
    bias = (b_ih + b_hh).reshape(1, -1)
    y_sorted = _lstm_call(xp, bo, msteps, degc, degr, w_ih, w_hh,
                          w_neigh_p, bias, npad)
    y_un = _sc_gather_rows(y_sorted, rankpad, ch=64)
    return _proj_call(xpad, y_un, w_self_p, b_self_p, act, npad)


def kernel(x, edge_index, W_ih1, W_hh1, b_ih1, b_hh1, fc_self_W1, fc_self_b1,
           fc_neigh_W1, W_ih2, W_hh2, b_ih2, b_hh2, fc_self_W2, fc_self_b2,
           fc_neigh_W2):
    n, d = x.shape
    e = edge_index.shape[1]
    npad = ((n + B - 1) // B) * B
    gran = NUM_WORKERS * 128
    epad = ((e + B + gran - 1) // gran) * gran

    pre = _prep(edge_index, n, npad, epad)
    xpad = jnp.zeros((npad, d), jnp.float32).at[:n].set(x)

    out1 = _layer(xpad, pre, W_ih1, W_hh1, b_ih1, b_hh1, fc_self_W1,
                  fc_self_b1, fc_neigh_W1, jax.nn.relu, npad, e, epad)
    out2 = _layer(out1, pre, W_ih2, W_hh2, b_ih2, b_hh2, fc_self_W2,
                  fc_self_b2, fc_neigh_W2, jax.nn.sigmoid, npad, e, epad)
    return out2[:n, :fc_self_W2.shape[0]]


# speedup vs baseline: 10.3241x; 1.2069x over previous
import functools

import jax
import jax.numpy as jnp
from jax import lax
from jax.experimental import pallas as pl
from jax.experimental.pallas import tpu as pltpu
from jax.experimental.pallas import tpu_sc as plsc

B = 512
NUM_WORKERS = 32


def _sc_gather_rows(table, idx, ch):
    m = idx.shape[0]
    _, d = table.shape
    per_w = m // NUM_WORKERS
    n_ch = per_w // ch
    assert per_w * NUM_WORKERS == m and n_ch * ch == per_w

    mesh = plsc.VectorSubcoreMesh(core_axis_name="c", subcore_axis_name="s")

    def body(table_hbm, idx_hbm, out_hbm, idx_v, buf, gsem):
        wid = lax.axis_index("s") * 2 + lax.axis_index("c")
        base = wid * per_w
        pltpu.sync_copy(idx_hbm.at[pl.ds(base, per_w)], idx_v)

        def chunk(i, carry):
            cp = pltpu.make_async_copy(
                table_hbm.at[idx_v.at[pl.ds(i * ch, ch)]], buf, gsem)
            cp.start()
            cp.wait()
            pltpu.sync_copy(buf, out_hbm.at[pl.ds(base + i * ch, ch)])
            return carry

        lax.fori_loop(0, n_ch, chunk, 0)

    f = pl.kernel(
        body,
        out_type=jax.ShapeDtypeStruct((m, d), jnp.float32),
        mesh=mesh,
        scratch_types=[
            pltpu.VMEM((per_w,), jnp.int32),
            pltpu.VMEM((ch, d), jnp.float32),
            pltpu.SemaphoreType.DMA,
        ],
    )
    return f(table, idx)


def _sc_pack_rows(table, src_s, dst_s, off_n, rank_n, f2_n, cs, degp, e, epad):
    _, d = table.shape
    n = off_n.shape[0]
    npad1 = cs.shape[0]
    ch = 128
    per_w = epad // NUM_WORKERS
    n_ch = per_w // ch
    assert n_ch * ch * NUM_WORKERS == epad

    mesh = plsc.VectorSubcoreMesh(core_axis_name="c", subcore_axis_name="s")

    def body(table_hbm, src_hbm, dst_hbm, off_hbm, rank_hbm, f2_hbm, cs_hbm,
             degp_hbm, xp_hbm, src_v, dst_v, off_v, rank_v, f2_v, cs_v,
             degp_v, pbuf, rows, gsem, ssem):
        wid = lax.axis_index("s") * 2 + lax.axis_index("c")
        base = wid * per_w
        pltpu.sync_copy(src_hbm.at[pl.ds(base, per_w)], src_v)
        pltpu.sync_copy(dst_hbm.at[pl.ds(base, per_w)], dst_v)
        pltpu.sync_copy(off_hbm, off_v)
        pltpu.sync_copy(rank_hbm, rank_v)
        pltpu.sync_copy(f2_hbm, f2_v)
        pltpu.sync_copy(cs_hbm, cs_v)
        pltpu.sync_copy(degp_hbm, degp_v)

        lane16 = lax.iota(jnp.int32, 16)

        def chunk(i, carry):
            for v in range(ch // 16):
                o = i * ch + v * 16
                j = base + o + lane16
                dstv = dst_v[pl.ds(o, 16)]
                offv = plsc.load_gather(off_v, [dstv])
                rkv = plsc.load_gather(rank_v, [dstv])
                f2v = plsc.load_gather(f2_v, [dstv])
                t = j - offv
                blkb = rkv & ~(B - 1)
                lo = jnp.zeros((16,), jnp.int32)
                hi = jnp.full((16,), B, jnp.int32)
                for _ in range((B - 1).bit_length()):
                    mid = (lo + hi) >> 1
                    dv = plsc.load_gather(degp_v, [blkb + mid])
                    ge = dv >= t
                    lo = jnp.where(ge, mid + 1, lo)
                    hi = jnp.where(ge, hi, mid)
                csv = plsc.load_gather(cs_v, [blkb + lo])
                p = f2v + t * lo - csv
                p = jnp.where(j < e, p, e + (j - e) % (epad - e))
                pbuf[pl.ds(v * 16, 16)] = p
            cp = pltpu.make_async_copy(
                table_hbm.at[src_v.at[pl.ds(i * ch, ch)]], rows, gsem)
            cp.start()
            cp.wait()
            sp = pltpu.make_async_copy(rows, xp_hbm.at[pbuf], ssem)
            sp.start()
            sp.wait()
            return carry

        lax.fori_loop(0, n_ch, chunk, 0)

    f = pl.kernel(
        body,
        out_type=jax.ShapeDtypeStruct((epad, d), jnp.float32),
        mesh=mesh,
        compiler_params=pltpu.CompilerParams(needs_layout_passes=False),
        scratch_types=[
            pltpu.VMEM((per_w,), jnp.int32),
            pltpu.VMEM((per_w,), jnp.int32),
            pltpu.VMEM((n,), jnp.int32),
            pltpu.VMEM((n,), jnp.int32),
            pltpu.VMEM((n,), jnp.int32),
            pltpu.VMEM((npad1,), jnp.int32),
            pltpu.VMEM((degp.shape[0],), jnp.int32),
            pltpu.VMEM((ch,), jnp.int32),
            pltpu.VMEM((ch, d), jnp.float32),
            pltpu.SemaphoreType.DMA,
            pltpu.SemaphoreType.DMA,
        ],
    )
    return f(table, src_s, dst_s, off_n, rank_n, f2_n, cs, degp)


def _lstm_body(bo_ref, ms_ref, degc_ref, degr_ref, wih_ref, whh_ref,
               fcn_ref, bias_ref, xp_ref, y_ref, h_ref, c_ref, slab_ref, sem):
    b = pl.program_id(0)
    n_steps = ms_ref[b]
    h_ref[...] = jnp.zeros_like(h_ref)
    c_ref[...] = jnp.zeros_like(c_ref)
    degc = degc_ref[...]
    degr = degr_ref[0]
    bias = bias_ref[...]
    hdim = h_ref.shape[1]
    s0 = bo_ref[b]

    @pl.when(n_steps > 0)
    def _():
        pltpu.make_async_copy(xp_ref.at[pl.ds(s0, B)], slab_ref.at[0],
                              sem.at[0]).start()

    def step(t, s):
        slot = lax.rem(t, 2)
        nslot = lax.rem(t + 1, 2)
        active = jnp.sum((t < degr).astype(jnp.int32))
        s_next = s + active

        @pl.when(t + 1 < n_steps)
        def _():
            pltpu.make_async_copy(xp_ref.at[pl.ds(s_next, B)],
                                  slab_ref.at[nslot], sem.at[nslot]).start()

        pltpu.make_async_copy(xp_ref.at[pl.ds(s, B)], slab_ref.at[slot],
                              sem.at[slot]).wait()
        xw = lax.dot_general(slab_ref[slot], wih_ref[...],
                             (((1,), (1,)), ((), ())),
                             preferred_element_type=jnp.float32)
        hw = lax.dot_general(h_ref[...], whh_ref[...],
                             (((1,), (1,)), ((), ())),
                             preferred_element_type=jnp.float32)
        gates = xw + hw + bias
        gi = jax.nn.sigmoid(gates[:, 0:hdim])
        gf = jax.nn.sigmoid(gates[:, hdim:2 * hdim])
        gg = jnp.tanh(gates[:, 2 * hdim:3 * hdim])
        go = jax.nn.sigmoid(gates[:, 3 * hdim:4 * hdim])
        c_new = gf * c_ref[...] + gi * gg
        h_new = go * jnp.tanh(c_new)
        m = t < degc
        h_ref[...] = jnp.where(m, h_new, h_ref[...])
        c_ref[...] = jnp.where(m, c_new, c_ref[...])
        return s_next

    lax.fori_loop(0, n_steps, step, s0)
    y_ref[...] = lax.dot_general(h_ref[...], fcn_ref[...],
                                 (((1,), (1,)), ((), ())),
                                 preferred_element_type=jnp.float32)


def _lstm_call(xp, bo, msteps, degc, degr, w_ih, w_hh, fcn, bias, npad):
    nb = npad // B
    h4 = w_ih.shape[0]
    d = w_ih.shape[1]
    hd = w_hh.shape[1]
    return pl.pallas_call(
        _lstm_body,
        grid=(nb,),
        in_specs=[
            pl.BlockSpec(memory_space=pltpu.SMEM),
            pl.BlockSpec(memory_space=pltpu.SMEM),
            pl.BlockSpec((B, 1), lambda b: (b, 0)),
            pl.BlockSpec((1, 1, B), lambda b: (b, 0, 0)),
            pl.BlockSpec((h4, d), lambda b: (0, 0)),
            pl.BlockSpec((h4, hd), lambda b: (0, 0)),
            pl.BlockSpec((hd, hd), lambda b: (0, 0)),
            pl.BlockSpec((1, h4), lambda b: (0, 0)),
            pl.BlockSpec(memory_space=pl.ANY),
        ],
        out_specs=pl.BlockSpec((B, hd), lambda b: (b, 0)),
        out_shape=jax.ShapeDtypeStruct((npad, hd), jnp.float32),
        scratch_shapes=[
            pltpu.VMEM((B, hd), jnp.float32),
            pltpu.VMEM((B, hd), jnp.float32),
            pltpu.VMEM((2, B, d), jnp.float32),
            pltpu.SemaphoreType.DMA((2,)),
        ],
    )(bo, msteps, degc, degr, w_ih, w_hh, fcn, bias, xp)


def _proj_body(x_ref, yun_ref, w_ref, b_ref, o_ref, *, act):
    o = lax.dot_general(x_ref[...], w_ref[...], (((1,), (1,)), ((), ())),
                        preferred_element_type=jnp.float32)
    o_ref[...] = act(o + b_ref[...] + yun_ref[...])


def _proj_call(xpad, yun, w, bias, act, npad):
    nb = npad // B
    d = w.shape[1]
    ho = w.shape[0]
    return pl.pallas_call(
        functools.partial(_proj_body, act=act),
        grid=(nb,),
        in_specs=[
            pl.BlockSpec((B, d), lambda b: (b, 0)),
            pl.BlockSpec((B, ho), lambda b: (b, 0)),
            pl.BlockSpec((ho, d), lambda b: (0, 0)),
            pl.BlockSpec((1, ho), lambda b: (0, 0)),
        ],
        out_specs=pl.BlockSpec((B, ho), lambda b: (b, 0)),
        out_shape=jax.ShapeDtypeStruct((npad, ho), jnp.float32),
    )(xpad, yun, w, bias)


def _prep(edge_index, n, npad, epad):
    src = edge_index[0]
    dst = edge_index[1]
    e = src.shape[0]
    deg = jnp.bincount(dst, length=n).astype(jnp.int32)
    offsets = (jnp.cumsum(deg) - deg).astype(jnp.int32)

    perm = jnp.argsort(-deg)
    degp = jnp.zeros(npad, jnp.int32).at[:n].set(deg[perm])
    rank = jnp.zeros(n, jnp.int32).at[perm].set(jnp.arange(n, dtype=jnp.int32))
    cs = jnp.concatenate([jnp.zeros(1, jnp.int32),
                          jnp.cumsum(degp).astype(jnp.int32)])
    blk_base = (rank // B) * B
    f2 = (cs[blk_base] + (rank - blk_base) + cs[blk_base + B]).astype(jnp.int32)

    order = jnp.argsort(dst)
    es = jnp.concatenate([src[:, None], dst[:, None]], axis=1)[order]
    src_s = jnp.zeros(epad, jnp.int32).at[:e].set(es[:, 0])
    dst_s = jnp.zeros(epad, jnp.int32).at[:e].set(es[:, 1])

    nb = npad // B
    bo = cs[jnp.arange(nb) * B]
    msteps = degp[jnp.arange(nb) * B]
    degc = degp.reshape(npad, 1)
    degr = degp.reshape(nb, 1, B)
    rankpad = jnp.zeros(npad, jnp.int32).at[:n].set(rank)
    return (src_s, dst_s, offsets, rank, f2, cs, degp,
            bo, msteps, degc, degr, rankpad)


def _layer(xpad, pre, w_ih, w_hh, b_ih, b_hh, w_self, b_self, w_neigh, act,
           npad, e, epad):
    (src_s, dst_s, offsets, rank, f2, cs, degp,
     bo, msteps, degc, degr, rankpad) = pre
    hd = w_hh.shape[1]
    ho = w_self.shape[0]
    w_self_p = jnp.zeros((hd, w_self.shape[1]), jnp.float32).at[:ho].set(w_self)
    b_self_p = jnp.zeros((1, hd), jnp.float32).at[0, :ho].set(b_self)
    w_neigh_p = jnp.zeros((hd, w_neigh.shape[1]), jnp.float32).at[:ho].set(w_neigh)

    xp = _sc_pack_rows(xpad, src_s, dst_s, offsets, rank, f2, cs, degp,
                       e, epad)
    bias = (b_ih + b_hh).reshape(1, -1)
    y_sorted = _lstm_call(xp, bo, msteps, degc, degr, w_ih, w_hh,
                          w_neigh_p, bias, npad)
    y_un = _sc_gather_rows(y_sorted, rankpad, ch=64)
    return _proj_call(xpad, y_un, w_self_p, b_self_p, act, npad)


def kernel(x, edge_index, W_ih1, W_hh1, b_ih1, b_hh1, fc_self_W1, fc_self_b1,
           fc_neigh_W1, W_ih2, W_hh2, b_ih2, b_hh2, fc_self_W2, fc_self_b2,
           fc_neigh_W2):
    n, d = x.shape
    e = edge_index.shape[1]
    npad = ((n + B - 1) // B) * B
    gran = NUM_WORKERS * 128
    epad = ((e + B + gran - 1) // gran) * gran

    pre = _prep(edge_index, n, npad, epad)
    xpad = jnp.zeros((npad, d), jnp.float32).at[:n].set(x)

    out1 = _layer(xpad, pre, W_ih1, W_hh1, b_ih1, b_hh1, fc_self_W1,
                  fc_self_b1, fc_neigh_W1, jax.nn.relu, npad, e, epad)
    out2 = _layer(out1, pre, W_ih2, W_hh2, b_ih2, b_hh2, fc_self_W2,
                  fc_self_b2, fc_neigh_W2, jax.nn.sigmoid, npad, e, epad)
    return out2[:n, :fc_self_W2.shape[0]]


# speedup vs baseline: 11.3826x; 1.1025x over previous
import functools

import jax
import jax.numpy as jnp
from jax import lax
from jax.experimental import pallas as pl
from jax.experimental.pallas import tpu as pltpu
from jax.experimental.pallas import tpu_sc as plsc

B = 1024
NUM_WORKERS = 32


def _sc_gather_rows(table, idx, ch):
    m = idx.shape[0]
    _, d = table.shape
    per_w = m // NUM_WORKERS
    n_ch = per_w // ch
    assert per_w * NUM_WORKERS == m and n_ch * ch == per_w

    mesh = plsc.VectorSubcoreMesh(core_axis_name="c", subcore_axis_name="s")

    def body(table_hbm, idx_hbm, out_hbm, idx_v, buf, gsem):
        wid = lax.axis_index("s") * 2 + lax.axis_index("c")
        base = wid * per_w
        pltpu.sync_copy(idx_hbm.at[pl.ds(base, per_w)], idx_v)

        def chunk(i, carry):
            cp = pltpu.make_async_copy(
                table_hbm.at[idx_v.at[pl.ds(i * ch, ch)]], buf, gsem)
            cp.start()
            cp.wait()
            pltpu.sync_copy(buf, out_hbm.at[pl.ds(base + i * ch, ch)])
            return carry

        lax.fori_loop(0, n_ch, chunk, 0)

    f = pl.kernel(
        body,
        out_type=jax.ShapeDtypeStruct((m, d), jnp.float32),
        mesh=mesh,
        scratch_types=[
            pltpu.VMEM((per_w,), jnp.int32),
            pltpu.VMEM((ch, d), jnp.float32),
            pltpu.SemaphoreType.DMA,
        ],
    )
    return f(table, idx)


def _sc_pack_rows(table, src_s, dst_s, off_n, rank_n, f2_n, cs, degp, e, epad):
    _, d = table.shape
    n = off_n.shape[0]
    npad1 = cs.shape[0]
    ch = 128
    per_w = epad // NUM_WORKERS
    n_ch = per_w // ch
    assert n_ch * ch * NUM_WORKERS == epad

    mesh = plsc.VectorSubcoreMesh(core_axis_name="c", subcore_axis_name="s")

    def body(table_hbm, src_hbm, dst_hbm, off_hbm, rank_hbm, f2_hbm, cs_hbm,
             degp_hbm, xp_hbm, src_v, dst_v, off_v, rank_v, f2_v, cs_v,
             degp_v, pbuf, rows, gsem, ssem):
        wid = lax.axis_index("s") * 2 + lax.axis_index("c")
        base = wid * per_w
        pltpu.sync_copy(src_hbm.at[pl.ds(base, per_w)], src_v)
        pltpu.sync_copy(dst_hbm.at[pl.ds(base, per_w)], dst_v)
        pltpu.sync_copy(off_hbm, off_v)
        pltpu.sync_copy(rank_hbm, rank_v)
        pltpu.sync_copy(f2_hbm, f2_v)
        pltpu.sync_copy(cs_hbm, cs_v)
        pltpu.sync_copy(degp_hbm, degp_v)

        lane16 = lax.iota(jnp.int32, 16)

        def chunk(i, carry):
            for v in range(ch // 16):
                o = i * ch + v * 16
                j = base + o + lane16
                dstv = dst_v[pl.ds(o, 16)]
                offv = plsc.load_gather(off_v, [dstv])
                rkv = plsc.load_gather(rank_v, [dstv])
                f2v = plsc.load_gather(f2_v, [dstv])
                t = j - offv
                blkb = rkv & ~(B - 1)
                lo = jnp.zeros((16,), jnp.int32)
                hi = jnp.full((16,), B, jnp.int32)
                for _ in range((B - 1).bit_length()):
                    mid = (lo + hi) >> 1
                    dv = plsc.load_gather(degp_v, [blkb + mid])
                    ge = dv >= t
                    lo = jnp.where(ge, mid + 1, lo)
                    hi = jnp.where(ge, hi, mid)
                csv = plsc.load_gather(cs_v, [blkb + lo])
                p = f2v + t * lo - csv
                p = jnp.where(j < e, p, e + (j - e) % (epad - e))
                pbuf[pl.ds(v * 16, 16)] = p
            cp = pltpu.make_async_copy(
                table_hbm.at[src_v.at[pl.ds(i * ch, ch)]], rows, gsem)
            cp.start()
            cp.wait()
            sp = pltpu.make_async_copy(rows, xp_hbm.at[pbuf], ssem)
            sp.start()
            sp.wait()
            return carry

        lax.fori_loop(0, n_ch, chunk, 0)

    f = pl.kernel(
        body,
        out_type=jax.ShapeDtypeStruct((epad, d), jnp.float32),
        mesh=mesh,
        compiler_params=pltpu.CompilerParams(needs_layout_passes=False),
        scratch_types=[
            pltpu.VMEM((per_w,), jnp.int32),
            pltpu.VMEM((per_w,), jnp.int32),
            pltpu.VMEM((n,), jnp.int32),
            pltpu.VMEM((n,), jnp.int32),
            pltpu.VMEM((n,), jnp.int32),
            pltpu.VMEM((npad1,), jnp.int32),
            pltpu.VMEM((degp.shape[0],), jnp.int32),
            pltpu.VMEM((ch,), jnp.int32),
            pltpu.VMEM((ch, d), jnp.float32),
            pltpu.SemaphoreType.DMA,
            pltpu.SemaphoreType.DMA,
        ],
    )
    return f(table, src_s, dst_s, off_n, rank_n, f2_n, cs, degp)


def _lstm_body(bo_ref, ms_ref, degc_ref, degr_ref, wih_ref, whh_ref,
               fcn_ref, bias_ref, xp_ref, y_ref, h_ref, c_ref, slab_ref, sem):
    b = pl.program_id(0)
    n_steps = ms_ref[b]
    h_ref[...] = jnp.zeros_like(h_ref)
    c_ref[...] = jnp.zeros_like(c_ref)
    degc = degc_ref[...]
    degr = degr_ref[0]
    bias = bias_ref[...]
    hdim = h_ref.shape[1]
    s0 = bo_ref[b]

    @pl.when(n_steps > 0)
    def _():
        pltpu.make_async_copy(xp_ref.at[pl.ds(s0, B)], slab_ref.at[0],
                              sem.at[0]).start()

    def step(t, s):
        slot = lax.rem(t, 2)
        nslot = lax.rem(t + 1, 2)
        active = jnp.sum((t < degr).astype(jnp.int32))
        s_next = s + active

        @pl.when(t + 1 < n_steps)
        def _():
            pltpu.make_async_copy(xp_ref.at[pl.ds(s_next, B)],
                                  slab_ref.at[nslot], sem.at[nslot]).start()

        pltpu.make_async_copy(xp_ref.at[pl.ds(s, B)], slab_ref.at[slot],
                              sem.at[slot]).wait()
        xw = lax.dot_general(slab_ref[slot], wih_ref[...],
                             (((1,), (1,)), ((), ())),
                             preferred_element_type=jnp.float32)
        hw = lax.dot_general(h_ref[...], whh_ref[...],
                             (((1,), (1,)), ((), ())),
                             preferred_element_type=jnp.float32)
        gates = xw + hw + bias
        gi = jax.nn.sigmoid(gates[:, 0:hdim])
        gf = jax.nn.sigmoid(gates[:, hdim:2 * hdim])
        gg = jnp.tanh(gates[:, 2 * hdim:3 * hdim])
        go = jax.nn.sigmoid(gates[:, 3 * hdim:4 * hdim])
        c_new = gf * c_ref[...] + gi * gg
        h_new = go * jnp.tanh(c_new)
        m = t < degc
        h_ref[...] = jnp.where(m, h_new, h_ref[...])
        c_ref[...] = jnp.where(m, c_new, c_ref[...])
        return s_next

    lax.fori_loop(0, n_steps, step, s0)
    y_ref[...] = lax.dot_general(h_ref[...], fcn_ref[...],
                                 (((1,), (1,)), ((), ())),
                                 preferred_element_type=jnp.float32)


def _lstm_call(xp, bo, msteps, degc, degr, w_ih, w_hh, fcn, bias, npad):
    nb = npad // B
    h4 = w_ih.shape[0]
    d = w_ih.shape[1]
    hd = w_hh.shape[1]
    return pl.pallas_call(
        _lstm_body,
        grid=(nb,),
        in_specs=[
            pl.BlockSpec(memory_space=pltpu.SMEM),
            pl.BlockSpec(memory_space=pltpu.SMEM),
            pl.BlockSpec((B, 1), lambda b: (b, 0)),
            pl.BlockSpec((1, 1, B), lambda b: (b, 0, 0)),
            pl.BlockSpec((h4, d), lambda b: (0, 0)),
            pl.BlockSpec((h4, hd), lambda b: (0, 0)),
            pl.BlockSpec((hd, hd), lambda b: (0, 0)),
            pl.BlockSpec((1, h4), lambda b: (0, 0)),
            pl.BlockSpec(memory_space=pl.ANY),
        ],
        out_specs=pl.BlockSpec((B, hd), lambda b: (b, 0)),
        out_shape=jax.ShapeDtypeStruct((npad, hd), jnp.float32),
        scratch_shapes=[
            pltpu.VMEM((B, hd), jnp.float32),
            pltpu.VMEM((B, hd), jnp.float32),
            pltpu.VMEM((2, B, d), jnp.float32),
            pltpu.SemaphoreType.DMA((2,)),
        ],
    )(bo, msteps, degc, degr, w_ih, w_hh, fcn, bias, xp)


def _proj_body(x_ref, yun_ref, w_ref, b_ref, o_ref, *, act):
    o = lax.dot_general(x_ref[...], w_ref[...], (((1,), (1,)), ((), ())),
                        preferred_element_type=jnp.float32)
    o_ref[...] = act(o + b_ref[...] + yun_ref[...])


def _proj_call(xpad, yun, w, bias, act, npad):
    nb = npad // B
    d = w.shape[1]
    ho = w.shape[0]
    return pl.pallas_call(
        functools.partial(_proj_body, act=act),
        grid=(nb,),
        in_specs=[
            pl.BlockSpec((B, d), lambda b: (b, 0)),
            pl.BlockSpec((B, ho), lambda b: (b, 0)),
            pl.BlockSpec((ho, d), lambda b: (0, 0)),
            pl.BlockSpec((1, ho), lambda b: (0, 0)),
        ],
        out_specs=pl.BlockSpec((B, ho), lambda b: (b, 0)),
        out_shape=jax.ShapeDtypeStruct((npad, ho), jnp.float32),
    )(xpad, yun, w, bias)


def _prep(edge_index, n, npad, epad):
    src = edge_index[0]
    dst = edge_index[1]
    e = src.shape[0]
    deg = jnp.bincount(dst, length=n).astype(jnp.int32)
    offsets = (jnp.cumsum(deg) - deg).astype(jnp.int32)

    perm = jnp.argsort(-deg)
    degp = jnp.zeros(npad, jnp.int32).at[:n].set(deg[perm])
    rank = jnp.zeros(n, jnp.int32).at[perm].set(jnp.arange(n, dtype=jnp.int32))
    cs = jnp.concatenate([jnp.zeros(1, jnp.int32),
                          jnp.cumsum(degp).astype(jnp.int32)])
    blk_base = (rank // B) * B
    f2 = (cs[blk_base] + (rank - blk_base) + cs[blk_base + B]).astype(jnp.int32)

    order = jnp.argsort(dst)
    es = jnp.concatenate([src[:, None], dst[:, None]], axis=1)[order]
    src_s = jnp.zeros(epad, jnp.int32).at[:e].set(es[:, 0])
    dst_s = jnp.zeros(epad, jnp.int32).at[:e].set(es[:, 1])

    nb = npad // B
    bo = cs[jnp.arange(nb) * B]
    msteps = degp[jnp.arange(nb) * B]
    degc = degp.reshape(npad, 1)
    degr = degp.reshape(nb, 1, B)
    rankpad = jnp.zeros(npad, jnp.int32).at[:n].set(rank)
    return (src_s, dst_s, offsets, rank, f2, cs, degp,
            bo, msteps, degc, degr, rankpad)


def _layer(xpad, pre, w_ih, w_hh, b_ih, b_hh, w_self, b_self, w_neigh, act,
           npad, e, epad):
    (src_s, dst_s, offsets, rank, f2, cs, degp,
     bo, msteps, degc, degr, rankpad) = pre
    hd = w_hh.shape[1]
    ho = w_self.shape[0]
    w_self_p = jnp.zeros((hd, w_self.shape[1]), jnp.float32).at[:ho].set(w_self)
    b_self_p = jnp.zeros((1, hd), jnp.float32).at[0, :ho].set(b_self)
    w_neigh_p = jnp.zeros((hd, w_neigh.shape[1]), jnp.float32).at[:ho].set(w_neigh)

    xp = _sc_pack_rows(xpad, src_s, dst_s, offsets, rank, f2, cs, degp,
                       e, epad)
    bias = (b_ih + b_hh).reshape(1, -1)
    y_sorted = _lstm_call(xp, bo, msteps, degc, degr, w_ih, w_hh,
                          w_neigh_p, bias, npad)
    y_un = _sc_gather_rows(y_sorted, rankpad, ch=64)
    return _proj_call(xpad, y_un, w_self_p, b_self_p, act, npad)


def kernel(x, edge_index, W_ih1, W_hh1, b_ih1, b_hh1, fc_self_W1, fc_self_b1,
           fc_neigh_W1, W_ih2, W_hh2, b_ih2, b_hh2, fc_self_W2, fc_self_b2,
           fc_neigh_W2):
    n, d = x.shape
    e = edge_index.shape[1]
    npad = ((n + B - 1) // B) * B
    gran = NUM_WORKERS * 128
    epad = ((e + B + gran - 1) // gran) * gran

    pre = _prep(edge_index, n, npad, epad)
    xpad = jnp.zeros((npad, d), jnp.float32).at[:n].set(x)

    out1 = _layer(xpad, pre, W_ih1, W_hh1, b_ih1, b_hh1, fc_self_W1,
                  fc_self_b1, fc_neigh_W1, jax.nn.relu, npad, e, epad)
    out2 = _layer(out1, pre, W_ih2, W_hh2, b_ih2, b_hh2, fc_self_W2,
                  fc_self_b2, fc_neigh_W2, jax.nn.sigmoid, npad, e, epad)
    return out2[:n, :fc_self_W2.shape[0]]


# speedup vs baseline: 11.5726x; 1.0167x over previous
import functools

import jax
import jax.numpy as jnp
from jax import lax
from jax.experimental import pallas as pl
from jax.experimental.pallas import tpu as pltpu
from jax.experimental.pallas import tpu_sc as plsc

B = 2048
NUM_WORKERS = 32


def _sc_gather_rows(table, idx, ch):
    m = idx.shape[0]
    _, d = table.shape
    per_w = m // NUM_WORKERS
    n_ch = per_w // ch
    assert per_w * NUM_WORKERS == m and n_ch * ch == per_w

    mesh = plsc.VectorSubcoreMesh(core_axis_name="c", subcore_axis_name="s")

    def body(table_hbm, idx_hbm, out_hbm, idx_v, buf, gsem):
        wid = lax.axis_index("s") * 2 + lax.axis_index("c")
        base = wid * per_w
        pltpu.sync_copy(idx_hbm.at[pl.ds(base, per_w)], idx_v)

        def chunk(i, carry):
            cp = pltpu.make_async_copy(
                table_hbm.at[idx_v.at[pl.ds(i * ch, ch)]], buf, gsem)
            cp.start()
            cp.wait()
            pltpu.sync_copy(buf, out_hbm.at[pl.ds(base + i * ch, ch)])
            return carry

        lax.fori_loop(0, n_ch, chunk, 0)

    f = pl.kernel(
        body,
        out_type=jax.ShapeDtypeStruct((m, d), jnp.float32),
        mesh=mesh,
        scratch_types=[
            pltpu.VMEM((per_w,), jnp.int32),
            pltpu.VMEM((ch, d), jnp.float32),
            pltpu.SemaphoreType.DMA,
        ],
    )
    return f(table, idx)


def _sc_pack_rows(table, src_s, dst_s, off_n, rank_n, f2_n, cs, degp, e, epad):
    _, d = table.shape
    n = off_n.shape[0]
    npad1 = cs.shape[0]
    ch = 128
    per_w = epad // NUM_WORKERS
    n_ch = per_w // ch
    assert n_ch * ch * NUM_WORKERS == epad

    mesh = plsc.VectorSubcoreMesh(core_axis_name="c", subcore_axis_name="s")

    def body(table_hbm, src_hbm, dst_hbm, off_hbm, rank_hbm, f2_hbm, cs_hbm,
             degp_hbm, xp_hbm, src_v, dst_v, off_v, rank_v, f2_v, cs_v,
             degp_v, pbuf, rows, gsem, ssem):
        wid = lax.axis_index("s") * 2 + lax.axis_index("c")
        base = wid * per_w
        pltpu.sync_copy(src_hbm.at[pl.ds(base, per_w)], src_v)
        pltpu.sync_copy(dst_hbm.at[pl.ds(base, per_w)], dst_v)
        pltpu.sync_copy(off_hbm, off_v)
        pltpu.sync_copy(rank_hbm, rank_v)
        pltpu.sync_copy(f2_hbm, f2_v)
        pltpu.sync_copy(cs_hbm, cs_v)
        pltpu.sync_copy(degp_hbm, degp_v)

        lane16 = lax.iota(jnp.int32, 16)

        def chunk(i, carry):
            for v in range(ch // 16):
                o = i * ch + v * 16
                j = base + o + lane16
                dstv = dst_v[pl.ds(o, 16)]
                offv = plsc.load_gather(off_v, [dstv])
                rkv = plsc.load_gather(rank_v, [dstv])
                f2v = plsc.load_gather(f2_v, [dstv])
                t = j - offv
                blkb = rkv & ~(B - 1)
                lo = jnp.zeros((16,), jnp.int32)
                hi = jnp.full((16,), B, jnp.int32)
                for _ in range((B - 1).bit_length()):
                    mid = (lo + hi) >> 1
                    dv = plsc.load_gather(degp_v, [blkb + mid])
                    ge = dv >= t
                    lo = jnp.where(ge, mid + 1, lo)
                    hi = jnp.where(ge, hi, mid)
                csv = plsc.load_gather(cs_v, [blkb + lo])
                p = f2v + t * lo - csv
                p = jnp.where(j < e, p, e + (j - e) % (epad - e))
                pbuf[pl.ds(v * 16, 16)] = p
            cp = pltpu.make_async_copy(
                table_hbm.at[src_v.at[pl.ds(i * ch, ch)]], rows, gsem)
            cp.start()
            cp.wait()
            sp = pltpu.make_async_copy(rows, xp_hbm.at[pbuf], ssem)
            sp.start()
            sp.wait()
            return carry

        lax.fori_loop(0, n_ch, chunk, 0)

    f = pl.kernel(
        body,
        out_type=jax.ShapeDtypeStruct((epad, d), jnp.float32),
        mesh=mesh,
        compiler_params=pltpu.CompilerParams(needs_layout_passes=False),
        scratch_types=[
            pltpu.VMEM((per_w,), jnp.int32),
            pltpu.VMEM((per_w,), jnp.int32),
            pltpu.VMEM((n,), jnp.int32),
            pltpu.VMEM((n,), jnp.int32),
            pltpu.VMEM((n,), jnp.int32),
            pltpu.VMEM((npad1,), jnp.int32),
            pltpu.VMEM((degp.shape[0],), jnp.int32),
            pltpu.VMEM((ch,), jnp.int32),
            pltpu.VMEM((ch, d), jnp.float32),
            pltpu.SemaphoreType.DMA,
            pltpu.SemaphoreType.DMA,
        ],
    )
    return f(table, src_s, dst_s, off_n, rank_n, f2_n, cs, degp)


def _lstm_body(bo_ref, ms_ref, degc_ref, degr_ref, wih_ref, whh_ref,
               fcn_ref, bias_ref, xp_ref, y_ref, h_ref, c_ref, slab_ref, sem):
    b = pl.program_id(0)
    n_steps = ms_ref[b]
    h_ref[...] = jnp.zeros_like(h_ref)
    c_ref[...] = jnp.zeros_like(c_ref)
    degc = degc_ref[...]
    degr = degr_ref[0]
    bias = bias_ref[...]
    hdim = h_ref.shape[1]
    s0 = bo_ref[b]

    @pl.when(n_steps > 0)
    def _():
        pltpu.make_async_copy(xp_ref.at[pl.ds(s0, B)], slab_ref.at[0],
                              sem.at[0]).start()

    def step(t, s):
        slot = lax.rem(t, 2)
        nslot = lax.rem(t + 1, 2)
        active = jnp.sum((t < degr).astype(jnp.int32))
        s_next = s + active

        @pl.when(t + 1 < n_steps)
        def _():
            pltpu.make_async_copy(xp_ref.at[pl.ds(s_next, B)],
                                  slab_ref.at[nslot], sem.at[nslot]).start()

        pltpu.make_async_copy(xp_ref.at[pl.ds(s, B)], slab_ref.at[slot],
                              sem.at[slot]).wait()
        xw = lax.dot_general(slab_ref[slot], wih_ref[...],
                             (((1,), (1,)), ((), ())),
                             preferred_element_type=jnp.float32)
        hw = lax.dot_general(h_ref[...], whh_ref[...],
                             (((1,), (1,)), ((), ())),
                             preferred_element_type=jnp.float32)
        gates = xw + hw + bias
        gi = jax.nn.sigmoid(gates[:, 0:hdim])
        gf = jax.nn.sigmoid(gates[:, hdim:2 * hdim])
        gg = jnp.tanh(gates[:, 2 * hdim:3 * hdim])
        go = jax.nn.sigmoid(gates[:, 3 * hdim:4 * hdim])
        c_new = gf * c_ref[...] + gi * gg
        h_new = go * jnp.tanh(c_new)
        m = t < degc
        h_ref[...] = jnp.where(m, h_new, h_ref[...])
        c_ref[...] = jnp.where(m, c_new, c_ref[...])
        return s_next

    lax.fori_loop(0, n_steps, step, s0)
    y_ref[...] = lax.dot_general(h_ref[...], fcn_ref[...],
                                 (((1,), (1,)), ((), ())),
                                 preferred_element_type=jnp.float32)


def _lstm_call(xp, bo, msteps, degc, degr, w_ih, w_hh, fcn, bias, npad):
    nb = npad // B
    h4 = w_ih.shape[0]
    d = w_ih.shape[1]
    hd = w_hh.shape[1]
    return pl.pallas_call(
        _lstm_body,
        grid=(nb,),
        in_specs=[
            pl.BlockSpec(memory_space=pltpu.SMEM),
            pl.BlockSpec(memory_space=pltpu.SMEM),
            pl.BlockSpec((B, 1), lambda b: (b, 0)),
            pl.BlockSpec((1, 1, B), lambda b: (b, 0, 0)),
            pl.BlockSpec((h4, d), lambda b: (0, 0)),
            pl.BlockSpec((h4, hd), lambda b: (0, 0)),
            pl.BlockSpec((hd, hd), lambda b: (0, 0)),
            pl.BlockSpec((1, h4), lambda b: (0, 0)),
            pl.BlockSpec(memory_space=pl.ANY),
        ],
        out_specs=pl.BlockSpec((B, hd), lambda b: (b, 0)),
        out_shape=jax.ShapeDtypeStruct((npad, hd), jnp.float32),
        scratch_shapes=[
            pltpu.VMEM((B, hd), jnp.float32),
            pltpu.VMEM((B, hd), jnp.float32),
            pltpu.VMEM((2, B, d), jnp.float32),
            pltpu.SemaphoreType.DMA((2,)),
        ],
    )(bo, msteps, degc, degr, w_ih, w_hh, fcn, bias, xp)


def _proj_body(x_ref, yun_ref, w_ref, b_ref, o_ref, *, act):
    o = lax.dot_general(x_ref[...], w_ref[...], (((1,), (1,)), ((), ())),
                        preferred_element_type=jnp.float32)
    o_ref[...] = act(o + b_ref[...] + yun_ref[...])


def _proj_call(xpad, yun, w, bias, act, npad):
    nb = npad // B
    d = w.shape[1]
    ho = w.shape[0]
    return pl.pallas_call(
        functools.partial(_proj_body, act=act),
        grid=(nb,),
        in_specs=[
            pl.BlockSpec((B, d), lambda b: (b, 0)),
            pl.BlockSpec((B, ho), lambda b: (b, 0)),
            pl.BlockSpec((ho, d), lambda b: (0, 0)),
            pl.BlockSpec((1, ho), lambda b: (0, 0)),
        ],
        out_specs=pl.BlockSpec((B, ho), lambda b: (b, 0)),
        out_shape=jax.ShapeDtypeStruct((npad, ho), jnp.float32),
    )(xpad, yun, w, bias)


def _prep(edge_index, n, npad, epad):
    src = edge_index[0]
    dst = edge_index[1]
    e = src.shape[0]
    deg = jnp.bincount(dst, length=n).astype(jnp.int32)
    offsets = (jnp.cumsum(deg) - deg).astype(jnp.int32)

    perm = jnp.argsort(-deg)
    degp = jnp.zeros(npad, jnp.int32).at[:n].set(deg[perm])
    rank = jnp.zeros(n, jnp.int32).at[perm].set(jnp.arange(n, dtype=jnp.int32))
    cs = jnp.concatenate([jnp.zeros(1, jnp.int32),
                          jnp.cumsum(degp).astype(jnp.int32)])
    blk_base = (rank // B) * B
    f2 = (cs[blk_base] + (rank - blk_base) + cs[blk_base + B]).astype(jnp.int32)

    order = jnp.argsort(dst)
    es = jnp.concatenate([src[:, None], dst[:, None]], axis=1)[order]
    src_s = jnp.zeros(epad, jnp.int32).at[:e].set(es[:, 0])
    dst_s = jnp.zeros(epad, jnp.int32).at[:e].set(es[:, 1])

    nb = npad // B
    bo = cs[jnp.arange(nb) * B]
    msteps = degp[jnp.arange(nb) * B]
    degc = degp.reshape(npad, 1)
    degr = degp.reshape(nb, 1, B)
    rankpad = jnp.zeros(npad, jnp.int32).at[:n].set(rank)
    return (src_s, dst_s, offsets, rank, f2, cs, degp,
            bo, msteps, degc, degr, rankpad)


def _layer(xpad, pre, w_ih, w_hh, b_ih, b_hh, w_self, b_self, w_neigh, act,
           npad, e, epad):
    (src_s, dst_s, offsets, rank, f2, cs, degp,
     bo, msteps, degc, degr, rankpad) = pre
    hd = w_hh.shape[1]
    ho = w_self.shape[0]
    w_self_p = jnp.zeros((hd, w_self.shape[1]), jnp.float32).at[:ho].set(w_self)
    b_self_p = jnp.zeros((1, hd), jnp.float32).at[0, :ho].set(b_self)
    w_neigh_p = jnp.zeros((hd, w_neigh.shape[1]), jnp.float32).at[:ho].set(w_neigh)

    xp = _sc_pack_rows(xpad, src_s, dst_s, offsets, rank, f2, cs, degp,
                       e, epad)
    bias = (b_ih + b_hh).reshape(1, -1)
    y_sorted = _lstm_call(xp, bo, msteps, degc, degr, w_ih, w_hh,
                          w_neigh_p, bias, npad)
    y_un = _sc_gather_rows(y_sorted, rankpad, ch=64)
    return _proj_call(xpad, y_un, w_self_p, b_self_p, act, npad)


def kernel(x, edge_index, W_ih1, W_hh1, b_ih1, b_hh1, fc_self_W1, fc_self_b1,
           fc_neigh_W1, W_ih2, W_hh2, b_ih2, b_hh2, fc_self_W2, fc_self_b2,
           fc_neigh_W2):
    n, d = x.shape
    e = edge_index.shape[1]
    npad = ((n + B - 1) // B) * B
    gran = NUM_WORKERS * 128
    epad = ((e + B + gran - 1) // gran) * gran

    pre = _prep(edge_index, n, npad, epad)
    xpad = jnp.zeros((npad, d), jnp.float32).at[:n].set(x)

    out1 = _layer(xpad, pre, W_ih1, W_hh1, b_ih1, b_hh1, fc_self_W1,
                  fc_self_b1, fc_neigh_W1, jax.nn.relu, npad, e, epad)
    out2 = _layer(out1, pre, W_ih2, W_hh2, b_ih2, b_hh2, fc_self_W2,
                  fc_self_b2, fc_neigh_W2, jax.nn.sigmoid, npad, e, epad)
    return out2[:n, :fc_self_W2.shape[0]]


# speedup vs baseline: 12.3716x; 1.0690x over previous
import functools

import jax
import jax.numpy as jnp
from jax import lax
from jax.experimental import pallas as pl
from jax.experimental.pallas import tpu as pltpu
from jax.experimental.pallas import tpu_sc as plsc

B = 2048
NUM_WORKERS = 32


def _sc_gather_rows(table, idx, ch):
    m = idx.shape[0]
    _, d = table.shape
    per_w = m // NUM_WORKERS
    n_ch = per_w // ch
    assert per_w * NUM_WORKERS == m and n_ch * ch == per_w

    mesh = plsc.VectorSubcoreMesh(core_axis_name="c", subcore_axis_name="s")

    def body(table_hbm, idx_hbm, out_hbm, idx_v, buf, gsem):
        wid = lax.axis_index("s") * 2 + lax.axis_index("c")
        base = wid * per_w
        pltpu.sync_copy(idx_hbm.at[pl.ds(base, per_w)], idx_v)

        def chunk(i, carry):
            cp = pltpu.make_async_copy(
                table_hbm.at[idx_v.at[pl.ds(i * ch, ch)]], buf, gsem)
            cp.start()
            cp.wait()
            pltpu.sync_copy(buf, out_hbm.at[pl.ds(base + i * ch, ch)])
            return carry

        lax.fori_loop(0, n_ch, chunk, 0)

    f = pl.kernel(
        body,
        out_type=jax.ShapeDtypeStruct((m, d), jnp.float32),
        mesh=mesh,
        scratch_types=[
            pltpu.VMEM((per_w,), jnp.int32),
            pltpu.VMEM((ch, d), jnp.float32),
            pltpu.SemaphoreType.DMA,
        ],
    )
    return f(table, idx)


def _sc_pack_rows(table, src_s, dst_s, off_n, rank_n, f2_n, cs, degp, e, epad):
    _, d = table.shape
    n = off_n.shape[0]
    npad1 = cs.shape[0]
    ch = 128
    per_w = epad // NUM_WORKERS
    n_ch = per_w // ch
    assert n_ch * ch * NUM_WORKERS == epad

    mesh = plsc.VectorSubcoreMesh(core_axis_name="c", subcore_axis_name="s")

    def body(table_hbm, src_hbm, dst_hbm, off_hbm, rank_hbm, f2_hbm, cs_hbm,
             degp_hbm, xp_hbm, src_v, dst_v, off_v, rank_v, f2_v, cs_v,
             degp_v, pbuf0, pbuf1, rows0, rows1, gsem0, gsem1, ssem0, ssem1):
        wid = lax.axis_index("s") * 2 + lax.axis_index("c")
        base = wid * per_w
        pltpu.sync_copy(src_hbm.at[pl.ds(base, per_w)], src_v)
        pltpu.sync_copy(dst_hbm.at[pl.ds(base, per_w)], dst_v)
        pltpu.sync_copy(off_hbm, off_v)
        pltpu.sync_copy(rank_hbm, rank_v)
        pltpu.sync_copy(f2_hbm, f2_v)
        pltpu.sync_copy(cs_hbm, cs_v)
        pltpu.sync_copy(degp_hbm, degp_v)

        lane16 = lax.iota(jnp.int32, 16)
        slots = ((pbuf0, rows0, gsem0, ssem0), (pbuf1, rows1, gsem1, ssem1))

        def compute_p(i, pb):
            for v in range(ch // 16):
                o = i * ch + v * 16
                j = base + o + lane16
                dstv = dst_v[pl.ds(o, 16)]
                offv = plsc.load_gather(off_v, [dstv])
                rkv = plsc.load_gather(rank_v, [dstv])
                f2v = plsc.load_gather(f2_v, [dstv])
                t = j - offv
                blkb = rkv & ~(B - 1)
                lo = jnp.zeros((16,), jnp.int32)
                hi = jnp.full((16,), B, jnp.int32)
                for _ in range((B - 1).bit_length()):
                    mid = (lo + hi) >> 1
                    dv = plsc.load_gather(degp_v, [blkb + mid])
                    ge = dv >= t
                    lo = jnp.where(ge, mid + 1, lo)
                    hi = jnp.where(ge, hi, mid)
                csv = plsc.load_gather(cs_v, [blkb + lo])
                p = f2v + t * lo - csv
                p = jnp.where(j < e, p, e + (j - e) % (epad - e))
                pb[pl.ds(v * 16, 16)] = p

        def gather_cp(i, rw, sm):
            return pltpu.make_async_copy(
                table_hbm.at[src_v.at[pl.ds(i * ch, ch)]], rw, sm)

        def scatter_cp(pb, rw, sm):
            return pltpu.make_async_copy(rw, xp_hbm.at[pb], sm)

        def stage(i, cur, nxt):
            pb, rw, gs, ss = cur
            npb, nrw, ngs, nss = nxt

            @pl.when(i >= 1)
            def _():
                scatter_cp(npb, nrw, nss).wait()

            @pl.when(i + 1 < n_ch)
            def _():
                compute_p(i + 1, npb)
                gather_cp(i + 1, nrw, ngs).start()

            gather_cp(i, rw, gs).wait()
            scatter_cp(pb, rw, ss).start()

        compute_p(0, pbuf0)
        gather_cp(0, rows0, gsem0).start()

        def pair(k, carry):
            stage(2 * k, slots[0], slots[1])
            stage(2 * k + 1, slots[1], slots[0])
            return carry

        assert n_ch % 2 == 0
        lax.fori_loop(0, n_ch // 2, pair, 0)
        last = slots[(n_ch - 1) % 2]
        scatter_cp(last[0], last[1], last[3]).wait()

    f = pl.kernel(
        body,
        out_type=jax.ShapeDtypeStruct((epad, d), jnp.float32),
        mesh=mesh,
        compiler_params=pltpu.CompilerParams(needs_layout_passes=False),
        scratch_types=[
            pltpu.VMEM((per_w,), jnp.int32),
            pltpu.VMEM((per_w,), jnp.int32),
            pltpu.VMEM((n,), jnp.int32),
            pltpu.VMEM((n,), jnp.int32),
            pltpu.VMEM((n,), jnp.int32),
            pltpu.VMEM((npad1,), jnp.int32),
            pltpu.VMEM((degp.shape[0],), jnp.int32),
            pltpu.VMEM((ch,), jnp.int32),
            pltpu.VMEM((ch,), jnp.int32),
            pltpu.VMEM((ch, d), jnp.float32),
            pltpu.VMEM((ch, d), jnp.float32),
            pltpu.SemaphoreType.DMA,
            pltpu.SemaphoreType.DMA,
            pltpu.SemaphoreType.DMA,
            pltpu.SemaphoreType.DMA,
        ],
    )
    return f(table, src_s, dst_s, off_n, rank_n, f2_n, cs, degp)


def _lstm_body(bo_ref, ms_ref, degc_ref, degr_ref, wih_ref, whh_ref,
               fcn_ref, bias_ref, xp_ref, y_ref, h_ref, c_ref, slab_ref, sem):
    b = pl.program_id(0)
    n_steps = ms_ref[b]
    h_ref[...] = jnp.zeros_like(h_ref)
    c_ref[...] = jnp.zeros_like(c_ref)
    degc = degc_ref[...]
    degr = degr_ref[0]
    bias = bias_ref[...]
    hdim = h_ref.shape[1]
    s0 = bo_ref[b]

    @pl.when(n_steps > 0)
    def _():
        pltpu.make_async_copy(xp_ref.at[pl.ds(s0, B)], slab_ref.at[0],
                              sem.at[0]).start()

    def step(t, s):
        slot = lax.rem(t, 2)
        nslot = lax.rem(t + 1, 2)
        active = jnp.sum((t < degr).astype(jnp.int32))
        s_next = s + active

        @pl.when(t + 1 < n_steps)
        def _():
            pltpu.make_async_copy(xp_ref.at[pl.ds(s_next, B)],
                                  slab_ref.at[nslot], sem.at[nslot]).start()

        pltpu.make_async_copy(xp_ref.at[pl.ds(s, B)], slab_ref.at[slot],
                              sem.at[slot]).wait()
        xw = lax.dot_general(slab_ref[slot], wih_ref[...],
                             (((1,), (1,)), ((), ())),
                             preferred_element_type=jnp.float32)
        hw = lax.dot_general(h_ref[...], whh_ref[...],
                             (((1,), (1,)), ((), ())),
                             preferred_element_type=jnp.float32)
        gates = xw + hw + bias
        gi = jax.nn.sigmoid(gates[:, 0:hdim])
        gf = jax.nn.sigmoid(gates[:, hdim:2 * hdim])
        gg = jnp.tanh(gates[:, 2 * hdim:3 * hdim])
        go = jax.nn.sigmoid(gates[:, 3 * hdim:4 * hdim])
        c_new = gf * c_ref[...] + gi * gg
        h_new = go * jnp.tanh(c_new)
        m = t < degc
        h_ref[...] = jnp.where(m, h_new, h_ref[...])
        c_ref[...] = jnp.where(m, c_new, c_ref[...])
        return s_next

    lax.fori_loop(0, n_steps, step, s0)
    y_ref[...] = lax.dot_general(h_ref[...], fcn_ref[...],
                                 (((1,), (1,)), ((), ())),
                                 preferred_element_type=jnp.float32)


def _lstm_call(xp, bo, msteps, degc, degr, w_ih, w_hh, fcn, bias, npad):
    nb = npad // B
    h4 = w_ih.shape[0]
    d = w_ih.shape[1]
    hd = w_hh.shape[1]
    return pl.pallas_call(
        _lstm_body,
        grid=(nb,),
        in_specs=[
            pl.BlockSpec(memory_space=pltpu.SMEM),
            pl.BlockSpec(memory_space=pltpu.SMEM),
            pl.BlockSpec((B, 1), lambda b: (b, 0)),
            pl.BlockSpec((1, 1, B), lambda b: (b, 0, 0)),
            pl.BlockSpec((h4, d), lambda b: (0, 0)),
            pl.BlockSpec((h4, hd), lambda b: (0, 0)),
            pl.BlockSpec((hd, hd), lambda b: (0, 0)),
            pl.BlockSpec((1, h4), lambda b: (0, 0)),
            pl.BlockSpec(memory_space=pl.ANY),
        ],
        out_specs=pl.BlockSpec((B, hd), lambda b: (b, 0)),
        out_shape=jax.ShapeDtypeStruct((npad, hd), jnp.float32),
        scratch_shapes=[
            pltpu.VMEM((B, hd), jnp.float32),
            pltpu.VMEM((B, hd), jnp.float32),
            pltpu.VMEM((2, B, d), jnp.float32),
            pltpu.SemaphoreType.DMA((2,)),
        ],
    )(bo, msteps, degc, degr, w_ih, w_hh, fcn, bias, xp)


def _proj_body(x_ref, yun_ref, w_ref, b_ref, o_ref, *, act):
    o = lax.dot_general(x_ref[...], w_ref[...], (((1,), (1,)), ((), ())),
                        preferred_element_type=jnp.float32)
    o_ref[...] = act(o + b_ref[...] + yun_ref[...])


def _proj_call(xpad, yun, w, bias, act, npad):
    nb = npad // B
    d = w.shape[1]
    ho = w.shape[0]
    return pl.pallas_call(
        functools.partial(_proj_body, act=act),
        grid=(nb,),
        in_specs=[
            pl.BlockSpec((B, d), lambda b: (b, 0)),
            pl.BlockSpec((B, ho), lambda b: (b, 0)),
            pl.BlockSpec((ho, d), lambda b: (0, 0)),
            pl.BlockSpec((1, ho), lambda b: (0, 0)),
        ],
        out_specs=pl.BlockSpec((B, ho), lambda b: (b, 0)),
        out_shape=jax.ShapeDtypeStruct((npad, ho), jnp.float32),
    )(xpad, yun, w, bias)


def _prep(edge_index, n, npad, epad):
    src = edge_index[0]
    dst = edge_index[1]
    e = src.shape[0]
    deg = jnp.bincount(dst, length=n).astype(jnp.int32)
    offsets = (jnp.cumsum(deg) - deg).astype(jnp.int32)

    perm = jnp.argsort(-deg)
    degp = jnp.zeros(npad, jnp.int32).at[:n].set(deg[perm])
    rank = jnp.zeros(n, jnp.int32).at[perm].set(jnp.arange(n, dtype=jnp.int32))
    cs = jnp.concatenate([jnp.zeros(1, jnp.int32),
                          jnp.cumsum(degp).astype(jnp.int32)])
    blk_base = (rank // B) * B
    f2 = (cs[blk_base] + (rank - blk_base) + cs[blk_base + B]).astype(jnp.int32)

    order = jnp.argsort(dst)
    es = jnp.concatenate([src[:, None], dst[:, None]], axis=1)[order]
    src_s = jnp.zeros(epad, jnp.int32).at[:e].set(es[:, 0])
    dst_s = jnp.zeros(epad, jnp.int32).at[:e].set(es[:, 1])

    nb = npad // B
    bo = cs[jnp.arange(nb) * B]
    msteps = degp[jnp.arange(nb) * B]
    degc = degp.reshape(npad, 1)
    degr = degp.reshape(nb, 1, B)
    rankpad = jnp.zeros(npad, jnp.int32).at[:n].set(rank)
    return (src_s, dst_s, offsets, rank, f2, cs, degp,
            bo, msteps, degc, degr, rankpad)


def _layer(xpad, pre, w_ih, w_hh, b_ih, b_hh, w_self, b_self, w_neigh, act,
           npad, e, epad):
    (src_s, dst_s, offsets, rank, f2, cs, degp,
     bo, msteps, degc, degr, rankpad) = pre
    hd = w_hh.shape[1]
    ho = w_self.shape[0]
    w_self_p = jnp.zeros((hd, w_self.shape[1]), jnp.float32).at[:ho].set(w_self)
    b_self_p = jnp.zeros((1, hd), jnp.float32).at[0, :ho].set(b_self)
    w_neigh_p = jnp.zeros((hd, w_neigh.shape[1]), jnp.float32).at[:ho].set(w_neigh)

    xp = _sc_pack_rows(xpad, src_s, dst_s, offsets, rank, f2, cs, degp,
                       e, epad)
    bias = (b_ih + b_hh).reshape(1, -1)
    y_sorted = _lstm_call(xp, bo, msteps, degc, degr, w_ih, w_hh,
                          w_neigh_p, bias, npad)
    y_un = _sc_gather_rows(y_sorted, rankpad, ch=64)
    return _proj_call(xpad, y_un, w_self_p, b_self_p, act, npad)


def kernel(x, edge_index, W_ih1, W_hh1, b_ih1, b_hh1, fc_self_W1, fc_self_b1,
           fc_neigh_W1, W_ih2, W_hh2, b_ih2, b_hh2, fc_self_W2, fc_self_b2,
           fc_neigh_W2):
    n, d = x.shape
    e = edge_index.shape[1]
    npad = ((n + B - 1) // B) * B
    gran = NUM_WORKERS * 128
    epad = ((e + B + gran - 1) // gran) * gran

    pre = _prep(edge_index, n, npad, epad)
    xpad = jnp.zeros((npad, d), jnp.float32).at[:n].set(x)

    out1 = _layer(xpad, pre, W_ih1, W_hh1, b_ih1, b_hh1, fc_self_W1,
                  fc_self_b1, fc_neigh_W1, jax.nn.relu, npad, e, epad)
    out2 = _layer(out1, pre, W_ih2, W_hh2, b_ih2, b_hh2, fc_self_W2,
                  fc_self_b2, fc_neigh_W2, jax.nn.sigmoid, npad, e, epad)
    return out2[:n, :fc_self_W2.shape[0]]


# speedup vs baseline: 12.4609x; 1.0072x over previous
import functools

import jax
import jax.numpy as jnp
from jax import lax
from jax.experimental import pallas as pl
from jax.experimental.pallas import tpu as pltpu
from jax.experimental.pallas import tpu_sc as plsc

B = 2048
NUM_WORKERS = 32


def _sc_gather_rows(table, idx, ch):
    m = idx.shape[0]
    _, d = table.shape
    per_w = m // NUM_WORKERS
    n_ch = per_w // ch
    assert per_w * NUM_WORKERS == m and n_ch * ch == per_w

    mesh = plsc.VectorSubcoreMesh(core_axis_name="c", subcore_axis_name="s")

    def body(table_hbm, idx_hbm, out_hbm, idx_v, buf, gsem):
        wid = lax.axis_index("s") * 2 + lax.axis_index("c")
        base = wid * per_w
        pltpu.sync_copy(idx_hbm.at[pl.ds(base, per_w)], idx_v)

        def chunk(i, carry):
            cp = pltpu.make_async_copy(
                table_hbm.at[idx_v.at[pl.ds(i * ch, ch)]], buf, gsem)
            cp.start()
            cp.wait()
            pltpu.sync_copy(buf, out_hbm.at[pl.ds(base + i * ch, ch)])
            return carry

        lax.fori_loop(0, n_ch, chunk, 0)

    f = pl.kernel(
        body,
        out_type=jax.ShapeDtypeStruct((m, d), jnp.float32),
        mesh=mesh,
        scratch_types=[
            pltpu.VMEM((per_w,), jnp.int32),
            pltpu.VMEM((ch, d), jnp.float32),
            pltpu.SemaphoreType.DMA,
        ],
    )
    return f(table, idx)


def _sc_pack_rows(table, src_s, dst_s, off_n, rank_n, f2_n, cs, degp, e, epad):
    _, d = table.shape
    n = off_n.shape[0]
    npad1 = cs.shape[0]
    ch = 128
    per_w = epad // NUM_WORKERS
    n_ch = per_w // ch
    assert n_ch * ch * NUM_WORKERS == epad

    mesh = plsc.VectorSubcoreMesh(core_axis_name="c", subcore_axis_name="s")

    def body(table_hbm, src_hbm, dst_hbm, off_hbm, rank_hbm, f2_hbm, cs_hbm,
             degp_hbm, xp_hbm, src_v, dst_v, off_v, rank_v, f2_v, cs_v,
             degp_v, pbuf0, pbuf1, pbuf2, rows0, rows1, rows2,
             gsem0, gsem1, gsem2, ssem0, ssem1, ssem2):
        wid = lax.axis_index("s") * 2 + lax.axis_index("c")
        base = wid * per_w
        pltpu.sync_copy(src_hbm.at[pl.ds(base, per_w)], src_v)
        pltpu.sync_copy(dst_hbm.at[pl.ds(base, per_w)], dst_v)
        pltpu.sync_copy(off_hbm, off_v)
        pltpu.sync_copy(rank_hbm, rank_v)
        pltpu.sync_copy(f2_hbm, f2_v)
        pltpu.sync_copy(cs_hbm, cs_v)
        pltpu.sync_copy(degp_hbm, degp_v)

        lane16 = lax.iota(jnp.int32, 16)
        slots = ((pbuf0, rows0, gsem0, ssem0), (pbuf1, rows1, gsem1, ssem1),
                 (pbuf2, rows2, gsem2, ssem2))

        def compute_p(i, pb):
            for v in range(ch // 16):
                o = i * ch + v * 16
                j = base + o + lane16
                dstv = dst_v[pl.ds(o, 16)]
                offv = plsc.load_gather(off_v, [dstv])
                rkv = plsc.load_gather(rank_v, [dstv])
                f2v = plsc.load_gather(f2_v, [dstv])
                t = j - offv
                blkb = rkv & ~(B - 1)
                lo = jnp.zeros((16,), jnp.int32)
                hi = jnp.full((16,), B, jnp.int32)
                for _ in range((B - 1).bit_length()):
                    mid = (lo + hi) >> 1
                    dv = plsc.load_gather(degp_v, [blkb + mid])
                    ge = dv >= t
                    lo = jnp.where(ge, mid + 1, lo)
                    hi = jnp.where(ge, hi, mid)
                csv = plsc.load_gather(cs_v, [blkb + lo])
                p = f2v + t * lo - csv
                p = jnp.where(j < e, p, e + (j - e) % (epad - e))
                pb[pl.ds(v * 16, 16)] = p

        def gather_cp(i, rw, sm):
            return pltpu.make_async_copy(
                table_hbm.at[src_v.at[pl.ds(i * ch, ch)]], rw, sm)

        def scatter_cp(pb, rw, sm):
            return pltpu.make_async_copy(rw, xp_hbm.at[pb], sm)

        ns = len(slots)

        def stage(i, cur, nxt):
            pb, rw, gs, ss = cur
            npb, nrw, ngs, nss = nxt

            @pl.when(i >= ns - 1)
            def _():
                scatter_cp(npb, nrw, nss).wait()

            @pl.when(i + 1 < n_ch)
            def _():
                compute_p(i + 1, npb)
                gather_cp(i + 1, nrw, ngs).start()

            gather_cp(i, rw, gs).wait()
            scatter_cp(pb, rw, ss).start()

        compute_p(0, slots[0][0])
        gather_cp(0, slots[0][1], slots[0][2]).start()

        def rotation(k, carry):
            for u in range(ns):
                stage(ns * k + u, slots[u], slots[(u + 1) % ns])
            return carry

        n_full = n_ch // ns
        lax.fori_loop(0, n_full, rotation, 0)
        for i in range(n_full * ns, n_ch):
            stage(i, slots[i % ns], slots[(i + 1) % ns])
        for i in range(max(n_ch - ns + 1, 0), n_ch):
            sl = slots[i % ns]
            scatter_cp(sl[0], sl[1], sl[3]).wait()

    f = pl.kernel(
        body,
        out_type=jax.ShapeDtypeStruct((epad, d), jnp.float32),
        mesh=mesh,
        compiler_params=pltpu.CompilerParams(needs_layout_passes=False),
        scratch_types=[
            pltpu.VMEM((per_w,), jnp.int32),
            pltpu.VMEM((per_w,), jnp.int32),
            pltpu.VMEM((n,), jnp.int32),
            pltpu.VMEM((n,), jnp.int32),
            pltpu.VMEM((n,), jnp.int32),
            pltpu.VMEM((npad1,), jnp.int32),
            pltpu.VMEM((degp.shape[0],), jnp.int32),
            pltpu.VMEM((ch,), jnp.int32),
            pltpu.VMEM((ch,), jnp.int32),
            pltpu.VMEM((ch,), jnp.int32),
            pltpu.VMEM((ch, d), jnp.float32),
            pltpu.VMEM((ch, d), jnp.float32),
            pltpu.VMEM((ch, d), jnp.float32),
            pltpu.SemaphoreType.DMA,
            pltpu.SemaphoreType.DMA,
            pltpu.SemaphoreType.DMA,
            pltpu.SemaphoreType.DMA,
            pltpu.SemaphoreType.DMA,
            pltpu.SemaphoreType.DMA,
        ],
    )
    return f(table, src_s, dst_s, off_n, rank_n, f2_n, cs, degp)


def _lstm_body(bo_ref, ms_ref, degc_ref, degr_ref, wih_ref, whh_ref,
               fcn_ref, bias_ref, xp_ref, y_ref, h_ref, c_ref, slab_ref, sem):
    b = pl.program_id(0)
    n_steps = ms_ref[b]
    h_ref[...] = jnp.zeros_like(h_ref)
    c_ref[...] = jnp.zeros_like(c_ref)
    degc = degc_ref[...]
    degr = degr_ref[0]
    bias = bias_ref[...]
    hdim = h_ref.shape[1]
    s0 = bo_ref[b]

    @pl.when(n_steps > 0)
    def _():
        pltpu.make_async_copy(xp_ref.at[pl.ds(s0, B)], slab_ref.at[0],
                              sem.at[0]).start()

    def step(t, s):
        slot = lax.rem(t, 2)
        nslot = lax.rem(t + 1, 2)
        active = jnp.sum((t < degr).astype(jnp.int32))
        s_next = s + active

        @pl.when(t + 1 < n_steps)
        def _():
            pltpu.make_async_copy(xp_ref.at[pl.ds(s_next, B)],
                                  slab_ref.at[nslot], sem.at[nslot]).start()

        pltpu.make_async_copy(xp_ref.at[pl.ds(s, B)], slab_ref.at[slot],
                              sem.at[slot]).wait()
        xw = lax.dot_general(slab_ref[slot], wih_ref[...],
                             (((1,), (1,)), ((), ())),
                             preferred_element_type=jnp.float32)
        hw = lax.dot_general(h_ref[...], whh_ref[...],
                             (((1,), (1,)), ((), ())),
                             preferred_element_type=jnp.float32)
        gates = xw + hw + bias
        gi = jax.nn.sigmoid(gates[:, 0:hdim])
        gf = jax.nn.sigmoid(gates[:, hdim:2 * hdim])
        gg = jnp.tanh(gates[:, 2 * hdim:3 * hdim])
        go = jax.nn.sigmoid(gates[:, 3 * hdim:4 * hdim])
        c_new = gf * c_ref[...] + gi * gg
        h_new = go * jnp.tanh(c_new)
        m = t < degc
        h_ref[...] = jnp.where(m, h_new, h_ref[...])
        c_ref[...] = jnp.where(m, c_new, c_ref[...])
        return s_next

    lax.fori_loop(0, n_steps, step, s0)
    y_ref[...] = lax.dot_general(h_ref[...], fcn_ref[...],
                                 (((1,), (1,)), ((), ())),
                                 preferred_element_type=jnp.float32)


def _lstm_call(xp, bo, msteps, degc, degr, w_ih, w_hh, fcn, bias, npad):
    nb = npad // B
    h4 = w_ih.shape[0]
    d = w_ih.shape[1]
    hd = w_hh.shape[1]
    return pl.pallas_call(
        _lstm_body,
        grid=(nb,),
        in_specs=[
            pl.BlockSpec(memory_space=pltpu.SMEM),
            pl.BlockSpec(memory_space=pltpu.SMEM),
            pl.BlockSpec((B, 1), lambda b: (b, 0)),
            pl.BlockSpec((1, 1, B), lambda b: (b, 0, 0)),
            pl.BlockSpec((h4, d), lambda b: (0, 0)),
            pl.BlockSpec((h4, hd), lambda b: (0, 0)),
            pl.BlockSpec((hd, hd), lambda b: (0, 0)),
            pl.BlockSpec((1, h4), lambda b: (0, 0)),
            pl.BlockSpec(memory_space=pl.ANY),
        ],
        out_specs=pl.BlockSpec((B, hd), lambda b: (b, 0)),
        out_shape=jax.ShapeDtypeStruct((npad, hd), jnp.float32),
        scratch_shapes=[
            pltpu.VMEM((B, hd), jnp.float32),
            pltpu.VMEM((B, hd), jnp.float32),
            pltpu.VMEM((2, B, d), jnp.float32),
            pltpu.SemaphoreType.DMA((2,)),
        ],
    )(bo, msteps, degc, degr, w_ih, w_hh, fcn, bias, xp)


def _proj_body(x_ref, yun_ref, w_ref, b_ref, o_ref, *, act):
    o = lax.dot_general(x_ref[...], w_ref[...], (((1,), (1,)), ((), ())),
                        preferred_element_type=jnp.float32)
    o_ref[...] = act(o + b_ref[...] + yun_ref[...])


def _proj_call(xpad, yun, w, bias, act, npad):
    nb = npad // B
    d = w.shape[1]
    ho = w.shape[0]
    return pl.pallas_call(
        functools.partial(_proj_body, act=act),
        grid=(nb,),
        in_specs=[
            pl.BlockSpec((B, d), lambda b: (b, 0)),
            pl.BlockSpec((B, ho), lambda b: (b, 0)),
            pl.BlockSpec((ho, d), lambda b: (0, 0)),
            pl.BlockSpec((1, ho), lambda b: (0, 0)),
        ],
        out_specs=pl.BlockSpec((B, ho), lambda b: (b, 0)),
        out_shape=jax.ShapeDtypeStruct((npad, ho), jnp.float32),
    )(xpad, yun, w, bias)


def _prep(edge_index, n, npad, epad):
    src = edge_index[0]
    dst = edge_index[1]
    e = src.shape[0]
    deg = jnp.bincount(dst, length=n).astype(jnp.int32)
    offsets = (jnp.cumsum(deg) - deg).astype(jnp.int32)

    perm = jnp.argsort(-deg)
    degp = jnp.zeros(npad, jnp.int32).at[:n].set(deg[perm])
    rank = jnp.zeros(n, jnp.int32).at[perm].set(jnp.arange(n, dtype=jnp.int32))
    cs = jnp.concatenate([jnp.zeros(1, jnp.int32),
                          jnp.cumsum(degp).astype(jnp.int32)])
    blk_base = (rank // B) * B
    f2 = (cs[blk_base] + (rank - blk_base) + cs[blk_base + B]).astype(jnp.int32)

    order = jnp.argsort(dst)
    es = jnp.concatenate([src[:, None], dst[:, None]], axis=1)[order]
    src_s = jnp.zeros(epad, jnp.int32).at[:e].set(es[:, 0])
    dst_s = jnp.zeros(epad, jnp.int32).at[:e].set(es[:, 1])

    nb = npad // B
    bo = cs[jnp.arange(nb) * B]
    msteps = degp[jnp.arange(nb) * B]
    degc = degp.reshape(npad, 1)
    degr = degp.reshape(nb, 1, B)
    rankpad = jnp.zeros(npad, jnp.int32).at[:n].set(rank)
    return (src_s, dst_s, offsets, rank, f2, cs, degp,
            bo, msteps, degc, degr, rankpad)


def _layer(xpad, pre, w_ih, w_hh, b_ih, b_hh, w_self, b_self, w_neigh, act,
           npad, e, epad):
    (src_s, dst_s, offsets, rank, f2, cs, degp,
     bo, msteps, degc, degr, rankpad) = pre
    hd = w_hh.shape[1]
    ho = w_self.shape[0]
    w_self_p = jnp.zeros((hd, w_self.shape[1]), jnp.float32).at[:ho].set(w_self)
    b_self_p = jnp.zeros((1, hd), jnp.float32).at[0, :ho].set(b_self)
    w_neigh_p = jnp.zeros((hd, w_neigh.shape[1]), jnp.float32).at[:ho].set(w_neigh)

    xp = _sc_pack_rows(xpad, src_s, dst_s, offsets, rank, f2, cs, degp,
                       e, epad)
    bias = (b_ih + b_hh).reshape(1, -1)
    y_sorted = _lstm_call(xp, bo, msteps, degc, degr, w_ih, w_hh,
                          w_neigh_p, bias, npad)
    y_un = _sc_gather_rows(y_sorted, rankpad, ch=64)
    return _proj_call(xpad, y_un, w_self_p, b_self_p, act, npad)


def kernel(x, edge_index, W_ih1, W_hh1, b_ih1, b_hh1, fc_self_W1, fc_self_b1,
           fc_neigh_W1, W_ih2, W_hh2, b_ih2, b_hh2, fc_self_W2, fc_self_b2,
           fc_neigh_W2):
    n, d = x.shape
    e = edge_index.shape[1]
    npad = ((n + B - 1) // B) * B
    gran = NUM_WORKERS * 128
    epad = ((e + B + gran - 1) // gran) * gran

    pre = _prep(edge_index, n, npad, epad)
    xpad = jnp.zeros((npad, d), jnp.float32).at[:n].set(x)

    out1 = _layer(xpad, pre, W_ih1, W_hh1, b_ih1, b_hh1, fc_self_W1,
                  fc_self_b1, fc_neigh_W1, jax.nn.relu, npad, e, epad)
    out2 = _layer(out1, pre, W_ih2, W_hh2, b_ih2, b_hh2, fc_self_W2,
                  fc_self_b2, fc_neigh_W2, jax.nn.sigmoid, npad, e, epad)
    return out2[:n, :fc_self_W2.shape[0]]
